# Initial kernel scaffold; baseline (speedup 1.0000x reference)
import jax
import jax.numpy as jnp
from jax.experimental import pallas as pl


def kernel(sentences, bushou, pinyin, weizhi, trans_weizhi, tags, lengths, emb1, emb2, Wq, bq, Wk, bk, Wv, bv, Wo, bo, Wih_f, Whh_f, bih_f, bhh_f, Wih_b, Whh_b, bih_b, bhh_b, Wout, bout, transitions, h0, c0):
    raise NotImplementedError("write your pallas kernel here")


# speedup vs baseline: 2.3343x; 2.3343x over previous
import functools

import jax
import jax.numpy as jnp
from jax.experimental import pallas as pl
from jax.experimental.pallas import tpu as pltpu

B, L, V = 64, 256, 8000
D_EMB = 100
D_MODEL = 256
H, DK = 4, 64
HID, HID2 = 256, 128
NT = 20
START, STOP = 18, 19
NEG = -1.0e30

_BBA = 8
_BHALF = B // 2


def _attn_kernel(fin_ref, wq_ref, wk_ref, wv_ref, wo_ref,
                 bq_ref, bk_ref, bv_ref, bo_ref, x_ref):
    wq = wq_ref[...]
    wk = wk_ref[...]
    wv = wv_ref[...]
    wo = wo_ref[...]
    for ii in range(_BBA):
        f = fin_ref[ii].astype(jnp.bfloat16)
        q = jnp.dot(f, wq, preferred_element_type=jnp.float32) + bq_ref[...]
        k = jnp.dot(f, wk, preferred_element_type=jnp.float32) + bk_ref[...]
        v = jnp.dot(f, wv, preferred_element_type=jnp.float32) + bv_ref[...]
        qb = q.astype(jnp.bfloat16)
        kb = k.astype(jnp.bfloat16)
        vb = v.astype(jnp.bfloat16)
        outs = []
        for h in range(H):
            sl = slice(h * DK, (h + 1) * DK)
            s = jax.lax.dot_general(
                qb[:, sl], kb[:, sl], (((1,), (1,)), ((), ())),
                preferred_element_type=jnp.float32) * 0.125
            m = jnp.max(s, axis=1, keepdims=True)
            e = jnp.exp(s - m)
            l = jnp.sum(e, axis=1, keepdims=True)
            o = jnp.dot(e.astype(jnp.bfloat16), vb[:, sl],
                        preferred_element_type=jnp.float32)
            outs.append(o / l)
        cat = jnp.concatenate(outs, axis=1)
        xo = jnp.dot(cat.astype(jnp.bfloat16), wo,
                     preferred_element_type=jnp.float32) + bo_ref[...]
        x_ref[ii] = xo.astype(jnp.bfloat16)


def _lstm_kernel(x_ref, wih_ref, whh_ref, bg_ref, wt_ref, h0_ref, c0_ref,
                 plog_ref, xg_ref):
    xall = x_ref[...].reshape(L * _BHALF, D_MODEL)
    xg = jnp.dot(xall, wih_ref[...],
                 preferred_element_type=jnp.float32) + bg_ref[...]
    xg_ref[...] = xg.astype(jnp.bfloat16).reshape(L, _BHALF, 8 * HID2)

    whf = whh_ref[0]
    whb = whh_ref[1]
    wtf = wt_ref[0]
    wtb = wt_ref[1]

    def sig(z):
        return 1.0 / (1.0 + jnp.exp(-z))

    def step(t, carry):
        hf, cf, hb, cb = carry
        tb = (L - 1) - t
        xgt_f = xg_ref[t]
        xgt_b = xg_ref[tb]
        gf = xgt_f[:, :4 * HID2].astype(jnp.float32) + jnp.dot(
            hf.astype(jnp.bfloat16), whf, preferred_element_type=jnp.float32)
        gb = xgt_b[:, 4 * HID2:].astype(jnp.float32) + jnp.dot(
            hb.astype(jnp.bfloat16), whb, preferred_element_type=jnp.float32)
        i_f = sig(gf[:, 0:HID2])
        f_f = sig(gf[:, HID2:2 * HID2])
        g_f = jnp.tanh(gf[:, 2 * HID2:3 * HID2])
        o_f = sig(gf[:, 3 * HID2:])
        cf = f_f * cf + i_f * g_f
        hf = o_f * jnp.tanh(cf)
        i_b = sig(gb[:, 0:HID2])
        f_b = sig(gb[:, HID2:2 * HID2])
        g_b = jnp.tanh(gb[:, 2 * HID2:3 * HID2])
        o_b = sig(gb[:, 3 * HID2:])
        cb = f_b * cb + i_b * g_b
        hb = o_b * jnp.tanh(cb)
        plog_ref[0, t] = jnp.dot(hf.astype(jnp.bfloat16), wtf,
                                 preferred_element_type=jnp.float32
                                 ).astype(jnp.bfloat16)
        plog_ref[1, tb] = jnp.dot(hb.astype(jnp.bfloat16), wtb,
                                  preferred_element_type=jnp.float32
                                  ).astype(jnp.bfloat16)
        return hf, cf, hb, cb

    init = (h0_ref[0], c0_ref[0], h0_ref[1], c0_ref[1])
    jax.lax.fori_loop(0, L, step, init)


def _crf_kernel(plog_ref, tag_ref, prev_ref, len_ref, trans_ref, tcs_ref,
                bout_ref, out_ref, logits_ref):
    logits_ref[...] = (plog_ref[0].astype(jnp.float32)
                       + plog_ref[1].astype(jnp.float32) + bout_ref[...])

    lane3 = jax.lax.broadcasted_iota(jnp.int32, (L, _BHALF, 128), 2)
    t3 = jax.lax.broadcasted_iota(jnp.int32, (L, _BHALF, 128), 0)
    tag3 = jnp.broadcast_to(tag_ref[...], (L, _BHALF, 128))
    prev3 = jnp.broadcast_to(prev_ref[...], (L, _BHALF, 128))
    len3 = jnp.broadcast_to(len_ref[...][None, :, :], (L, _BHALF, 128))

    curoh = jnp.where(lane3 == tag3, 1.0, 0.0)
    prevoh = jnp.where(lane3 == prev3, 1.0, 0.0)
    maskf = jnp.where(t3 < len3, 1.0, 0.0)

    transb = trans_ref[...].astype(jnp.bfloat16)
    rowvals = jnp.dot(
        prevoh.astype(jnp.bfloat16).reshape(L * _BHALF, 128), transb,
        preferred_element_type=jnp.float32).reshape(L, _BHALF, 128)
    lastsel = jnp.where(t3 == (len3 - 1), 1.0, 0.0)
    acc3 = ((logits_ref[...] + rowvals) * curoh * maskf
            + lastsel * curoh * tcs_ref[...][None, :, :])
    realb = jnp.sum(jnp.sum(acc3, axis=2, keepdims=True), axis=0)

    eb = jnp.exp(trans_ref[...]).astype(jnp.bfloat16)
    lane2 = jax.lax.broadcasted_iota(jnp.int32, (_BHALF, 128), 1)
    prev0 = jnp.where(lane2 < NT, 0.0, NEG)
    lenb = len_ref[...]

    def step(t, prv):
        lg = logits_ref[t]
        m = jnp.max(prv, axis=1, keepdims=True)
        p = jnp.exp(prv - m).astype(jnp.bfloat16)
        s = jnp.dot(p, eb, preferred_element_type=jnp.float32)
        new = jnp.maximum(jnp.log(s), NEG) + m + lg
        return jnp.where(t < lenb, new, prv)

    prv = jax.lax.fori_loop(0, L, step, prev0)
    v = prv + tcs_ref[...]
    m2 = jnp.max(v, axis=1, keepdims=True)
    tot = jnp.log(jnp.sum(jnp.exp(v - m2), axis=1, keepdims=True)) + m2
    part = jnp.sum(tot - realb)
    out_ref[...] = jnp.zeros((1, 1, 128), jnp.float32) + part


def kernel(sentences, bushou, pinyin, weizhi, trans_weizhi, tags, lengths,
           emb1, emb2, Wq, bq, Wk, bk, Wv, bv, Wo, bo,
           Wih_f, Whh_f, bih_f, bhh_f, Wih_b, Whh_b, bih_b, bhh_b,
           Wout, bout, transitions, h0, c0):
    f32 = jnp.float32
    bf16 = jnp.bfloat16
    e1 = emb1[sentences]
    e2 = emb2[bushou]
    fin = jnp.concatenate([e1, e2, pinyin, trans_weizhi, weizhi],
                          axis=2).astype(f32)

    wq = Wq.T.astype(bf16)
    wk = Wk.T.astype(bf16)
    wv = Wv.T.astype(bf16)
    wo = Wo.T.astype(bf16)
    bq_r = bq.reshape(1, D_MODEL).astype(f32)
    bk_r = bk.reshape(1, D_MODEL).astype(f32)
    bv_r = bv.reshape(1, D_MODEL).astype(f32)
    bo_r = bo.reshape(1, D_MODEL).astype(f32)

    nblk = B // _BBA
    x_bm = pl.pallas_call(
        _attn_kernel,
        grid=(2, nblk // 2),
        in_specs=[
            pl.BlockSpec((_BBA, L, D_MODEL),
                         lambda c, i: (c * (nblk // 2) + i, 0, 0)),
            pl.BlockSpec((D_MODEL, D_MODEL), lambda c, i: (0, 0)),
            pl.BlockSpec((D_MODEL, D_MODEL), lambda c, i: (0, 0)),
            pl.BlockSpec((D_MODEL, D_MODEL), lambda c, i: (0, 0)),
            pl.BlockSpec((D_MODEL, D_MODEL), lambda c, i: (0, 0)),
            pl.BlockSpec((1, D_MODEL), lambda c, i: (0, 0)),
            pl.BlockSpec((1, D_MODEL), lambda c, i: (0, 0)),
            pl.BlockSpec((1, D_MODEL), lambda c, i: (0, 0)),
            pl.BlockSpec((1, D_MODEL), lambda c, i: (0, 0)),
        ],
        out_specs=pl.BlockSpec((_BBA, L, D_MODEL),
                               lambda c, i: (c * (nblk // 2) + i, 0, 0)),
        out_shape=jax.ShapeDtypeStruct((B, L, D_MODEL), bf16),
        compiler_params=pltpu.CompilerParams(
            dimension_semantics=("parallel", "parallel")),
        name="attn_fused",
    )(fin, wq, wk, wv, wo, bq_r, bk_r, bv_r, bo_r)

    x_tm = x_bm.transpose(1, 0, 2)

    wih = jnp.concatenate([Wih_f.T, Wih_b.T], axis=1).astype(bf16)
    whh = jnp.stack([Whh_f.T, Whh_b.T]).astype(bf16)
    bg = jnp.concatenate([(bih_f + bhh_f), (bih_b + bhh_b)]
                         ).reshape(1, 8 * HID2).astype(f32)
    woutp = jnp.zeros((HID, 128), f32).at[:, :NT].set(Wout.T.astype(f32))
    wt = jnp.stack([woutp[:HID2], woutp[HID2:]]).astype(bf16)

    plog = pl.pallas_call(
        _lstm_kernel,
        grid=(2,),
        in_specs=[
            pl.BlockSpec((L, _BHALF, D_MODEL), lambda i: (0, i, 0)),
            pl.BlockSpec((D_MODEL, 8 * HID2), lambda i: (0, 0)),
            pl.BlockSpec((2, HID2, 4 * HID2), lambda i: (0, 0, 0)),
            pl.BlockSpec((1, 8 * HID2), lambda i: (0, 0)),
            pl.BlockSpec((2, HID2, 128), lambda i: (0, 0, 0)),
            pl.BlockSpec((2, _BHALF, HID2), lambda i: (0, i, 0)),
            pl.BlockSpec((2, _BHALF, HID2), lambda i: (0, i, 0)),
        ],
        out_specs=pl.BlockSpec((2, L, _BHALF, 128), lambda i: (0, 0, i, 0)),
        out_shape=jax.ShapeDtypeStruct((2, L, B, 128), bf16),
        scratch_shapes=[pltpu.VMEM((L, _BHALF, 8 * HID2), bf16)],
        compiler_params=pltpu.CompilerParams(
            dimension_semantics=("parallel",),
            vmem_limit_bytes=50 * 1024 * 1024),
        name="bilstm",
    )(x_tm, wih, whh, bg, wt, h0.astype(f32), c0.astype(f32))

    trans_pad = jnp.full((128, 128), NEG, f32).at[:NT, :NT].set(
        transitions.astype(f32))
    tcs = jnp.full((1, 128), NEG, f32).at[0, :NT].set(
        transitions[:, STOP].astype(f32))
    bout_r = jnp.zeros((1, 128), f32).at[0, :NT].set(bout.astype(f32))
    tagc = tags.T.astype(jnp.int32)[:, :, None]
    lab_prev = jnp.concatenate(
        [jnp.full((B, 1), START, tags.dtype), tags[:, :-1]], axis=1)
    prevc = lab_prev.T.astype(jnp.int32)[:, :, None]
    len_c = lengths.astype(jnp.int32)[:, None]

    parts = pl.pallas_call(
        _crf_kernel,
        grid=(2,),
        in_specs=[
            pl.BlockSpec((2, L, _BHALF, 128), lambda i: (0, 0, i, 0)),
            pl.BlockSpec((L, _BHALF, 1), lambda i: (0, i, 0)),
            pl.BlockSpec((L, _BHALF, 1), lambda i: (0, i, 0)),
            pl.BlockSpec((_BHALF, 1), lambda i: (i, 0)),
            pl.BlockSpec((128, 128), lambda i: (0, 0)),
            pl.BlockSpec((1, 128), lambda i: (0, 0)),
            pl.BlockSpec((1, 128), lambda i: (0, 0)),
        ],
        out_specs=pl.BlockSpec((1, 1, 128), lambda i: (i, 0, 0)),
        out_shape=jax.ShapeDtypeStruct((2, 1, 128), f32),
        scratch_shapes=[pltpu.VMEM((L, _BHALF, 128), f32)],
        compiler_params=pltpu.CompilerParams(
            dimension_semantics=("parallel",),
            vmem_limit_bytes=50 * 1024 * 1024),
        name="crf_nll",
    )(plog, tagc, prevc, len_c, trans_pad, tcs, bout_r)

    return parts[0, 0, 0] + parts[1, 0, 0]


# speedup vs baseline: 2.7919x; 1.1961x over previous
import functools

import jax
import jax.numpy as jnp
from jax.experimental import pallas as pl
from jax.experimental.pallas import tpu as pltpu

B, L, V = 64, 256, 8000
D_EMB = 100
D_MODEL = 256
H, DK = 4, 64
HID, HID2 = 256, 128
NT = 20
START, STOP = 18, 19
NEG = -1.0e30

_BBA = 8
_BHALF = B // 2


def _attn_kernel(fin_ref, wq_ref, wk_ref, wv_ref, wo_ref,
                 bq_ref, bk_ref, bv_ref, bo_ref, x_ref):
    wq = wq_ref[...]
    wk = wk_ref[...]
    wv = wv_ref[...]
    wo = wo_ref[...]
    for ii in range(_BBA):
        f = fin_ref[ii].astype(jnp.bfloat16)
        q = jnp.dot(f, wq, preferred_element_type=jnp.float32) + bq_ref[...]
        k = jnp.dot(f, wk, preferred_element_type=jnp.float32) + bk_ref[...]
        v = jnp.dot(f, wv, preferred_element_type=jnp.float32) + bv_ref[...]
        qb = q.astype(jnp.bfloat16)
        kb = k.astype(jnp.bfloat16)
        vb = v.astype(jnp.bfloat16)
        outs = []
        for h in range(H):
            sl = slice(h * DK, (h + 1) * DK)
            s = jax.lax.dot_general(
                qb[:, sl], kb[:, sl], (((1,), (1,)), ((), ())),
                preferred_element_type=jnp.float32) * 0.125
            m = jnp.max(s, axis=1, keepdims=True)
            e = jnp.exp(s - m)
            l = jnp.sum(e, axis=1, keepdims=True)
            o = jnp.dot(e.astype(jnp.bfloat16), vb[:, sl],
                        preferred_element_type=jnp.float32)
            outs.append(o / l)
        cat = jnp.concatenate(outs, axis=1)
        xo = jnp.dot(cat.astype(jnp.bfloat16), wo,
                     preferred_element_type=jnp.float32) + bo_ref[...]
        x_ref[ii] = xo.astype(jnp.bfloat16)


def _lstm_kernel(x_ref, wih_ref, wcat_ref, bg_ref, h0_ref, c0_ref,
                 plog_ref, xg_ref):
    xall = x_ref[...].reshape(L * _BHALF, D_MODEL)
    xg = jnp.dot(xall, wih_ref[...],
                 preferred_element_type=jnp.float32) + bg_ref[...]
    xg_ref[...] = xg.astype(jnp.bfloat16).reshape(L, _BHALF, 8 * HID2)

    wf = wcat_ref[0]
    wb = wcat_ref[1]
    G4 = 4 * HID2

    def sig(z):
        return 1.0 / (1.0 + jnp.exp(-z))

    def step(t, carry):
        hf, cf, hb, cb = carry
        tb = (L - 1) - t
        yf = jnp.dot(hf.astype(jnp.bfloat16), wf,
                     preferred_element_type=jnp.float32)
        yb = jnp.dot(hb.astype(jnp.bfloat16), wb,
                     preferred_element_type=jnp.float32)
        plog_ref[0, jnp.maximum(t - 1, 0)] = yf[:, G4:].astype(jnp.bfloat16)
        plog_ref[1, jnp.minimum(tb + 1, L - 1)] = yb[:, G4:].astype(
            jnp.bfloat16)
        gf = xg_ref[t][:, :G4].astype(jnp.float32) + yf[:, :G4]
        gb = xg_ref[tb][:, G4:].astype(jnp.float32) + yb[:, :G4]
        i_f = sig(gf[:, 0:HID2])
        f_f = sig(gf[:, HID2:2 * HID2])
        g_f = jnp.tanh(gf[:, 2 * HID2:3 * HID2])
        o_f = sig(gf[:, 3 * HID2:])
        cf = f_f * cf + i_f * g_f
        hf = o_f * jnp.tanh(cf)
        i_b = sig(gb[:, 0:HID2])
        f_b = sig(gb[:, HID2:2 * HID2])
        g_b = jnp.tanh(gb[:, 2 * HID2:3 * HID2])
        o_b = sig(gb[:, 3 * HID2:])
        cb = f_b * cb + i_b * g_b
        hb = o_b * jnp.tanh(cb)
        return hf, cf, hb, cb

    init = (h0_ref[0], c0_ref[0], h0_ref[1], c0_ref[1])
    hf, cf, hb, cb = jax.lax.fori_loop(0, L, step, init, unroll=4)
    plog_ref[0, L - 1] = jnp.dot(
        hf.astype(jnp.bfloat16), wf[:, G4:],
        preferred_element_type=jnp.float32).astype(jnp.bfloat16)
    plog_ref[1, 0] = jnp.dot(
        hb.astype(jnp.bfloat16), wb[:, G4:],
        preferred_element_type=jnp.float32).astype(jnp.bfloat16)


def _crf_kernel(plog_ref, tag_ref, prev_ref, len_ref, trans_ref, tcs_ref,
                bout_ref, out_ref, logits_ref, g2_ref):
    logits_ref[...] = (plog_ref[0].astype(jnp.float32)
                       + plog_ref[1].astype(jnp.float32) + bout_ref[...])

    lane3 = jax.lax.broadcasted_iota(jnp.int32, (L, _BHALF, 128), 2)
    t3 = jax.lax.broadcasted_iota(jnp.int32, (L, _BHALF, 128), 0)
    tag3 = jnp.broadcast_to(tag_ref[...], (L, _BHALF, 128))
    prev3 = jnp.broadcast_to(prev_ref[...], (L, _BHALF, 128))
    len3 = jnp.broadcast_to(len_ref[...][None, :, :], (L, _BHALF, 128))

    curoh = jnp.where(lane3 == tag3, 1.0, 0.0)
    prevoh = jnp.where(lane3 == prev3, 1.0, 0.0)
    maskf = jnp.where(t3 < len3, 1.0, 0.0)

    transb = trans_ref[...].astype(jnp.bfloat16)
    rowvals = jnp.dot(
        prevoh.astype(jnp.bfloat16).reshape(L * _BHALF, 128), transb,
        preferred_element_type=jnp.float32).reshape(L, _BHALF, 128)
    lastsel = jnp.where(t3 == (len3 - 1), 1.0, 0.0)
    acc3 = ((logits_ref[...] + rowvals) * curoh * maskf
            + lastsel * curoh * tcs_ref[...][None, :, :])
    realb = jnp.sum(jnp.sum(acc3, axis=2, keepdims=True), axis=0)

    eb = jnp.exp(trans_ref[...]).astype(jnp.bfloat16)
    lane2 = jax.lax.broadcasted_iota(jnp.int32, (_BHALF, 128), 1)
    prev0 = jnp.where(lane2 < NT, 0.0, NEG)
    lenb = len_ref[...]

    tmx = jnp.max(jnp.max(trans_ref[...], axis=1, keepdims=True),
                  axis=0, keepdims=True)
    gm = jnp.max(logits_ref[...], axis=2, keepdims=True)
    gs = jnp.maximum(gm + (tmx + 2.995732273553991)[None, :, :], 0.0)
    gprev = jnp.concatenate(
        [jnp.zeros((1, _BHALF, 1), jnp.float32), gs[:-1]], axis=0)
    g2_ref[...] = gs + gprev

    def step(t, carry):
        prv, m1, m0 = carry
        lg = logits_ref[t]
        mu = m0 + g2_ref[t]
        p = jnp.exp(prv - mu).astype(jnp.bfloat16)
        s = jnp.dot(p, eb, preferred_element_type=jnp.float32)
        new = jnp.maximum(jnp.log(s), NEG) + mu + lg
        prvn = jnp.where(t < lenb, new, prv)
        m1n = jnp.max(prvn, axis=1, keepdims=True)
        return prvn, m1n, m1

    m_init = jnp.zeros((_BHALF, 1), jnp.float32)
    prv, _, _ = jax.lax.fori_loop(0, L, step, (prev0, m_init, m_init),
                                  unroll=4)
    v = prv + tcs_ref[...]
    m2 = jnp.max(v, axis=1, keepdims=True)
    tot = jnp.log(jnp.sum(jnp.exp(v - m2), axis=1, keepdims=True)) + m2
    part = jnp.sum(tot - realb)
    out_ref[...] = jnp.zeros((1, 1, 128), jnp.float32) + part


def kernel(sentences, bushou, pinyin, weizhi, trans_weizhi, tags, lengths,
           emb1, emb2, Wq, bq, Wk, bk, Wv, bv, Wo, bo,
           Wih_f, Whh_f, bih_f, bhh_f, Wih_b, Whh_b, bih_b, bhh_b,
           Wout, bout, transitions, h0, c0):
    f32 = jnp.float32
    bf16 = jnp.bfloat16
    e1 = emb1[sentences]
    e2 = emb2[bushou]
    fin = jnp.concatenate([e1, e2, pinyin, trans_weizhi, weizhi],
                          axis=2).astype(f32)

    wq = Wq.T.astype(bf16)
    wk = Wk.T.astype(bf16)
    wv = Wv.T.astype(bf16)
    wo = Wo.T.astype(bf16)
    bq_r = bq.reshape(1, D_MODEL).astype(f32)
    bk_r = bk.reshape(1, D_MODEL).astype(f32)
    bv_r = bv.reshape(1, D_MODEL).astype(f32)
    bo_r = bo.reshape(1, D_MODEL).astype(f32)

    nblk = B // _BBA
    x_bm = pl.pallas_call(
        _attn_kernel,
        grid=(2, nblk // 2),
        in_specs=[
            pl.BlockSpec((_BBA, L, D_MODEL),
                         lambda c, i: (c * (nblk // 2) + i, 0, 0)),
            pl.BlockSpec((D_MODEL, D_MODEL), lambda c, i: (0, 0)),
            pl.BlockSpec((D_MODEL, D_MODEL), lambda c, i: (0, 0)),
            pl.BlockSpec((D_MODEL, D_MODEL), lambda c, i: (0, 0)),
            pl.BlockSpec((D_MODEL, D_MODEL), lambda c, i: (0, 0)),
            pl.BlockSpec((1, D_MODEL), lambda c, i: (0, 0)),
            pl.BlockSpec((1, D_MODEL), lambda c, i: (0, 0)),
            pl.BlockSpec((1, D_MODEL), lambda c, i: (0, 0)),
            pl.BlockSpec((1, D_MODEL), lambda c, i: (0, 0)),
        ],
        out_specs=pl.BlockSpec((_BBA, L, D_MODEL),
                               lambda c, i: (c * (nblk // 2) + i, 0, 0)),
        out_shape=jax.ShapeDtypeStruct((B, L, D_MODEL), bf16),
        compiler_params=pltpu.CompilerParams(
            dimension_semantics=("parallel", "parallel")),
        name="attn_fused",
    )(fin, wq, wk, wv, wo, bq_r, bk_r, bv_r, bo_r)

    x_tm = x_bm.transpose(1, 0, 2)

    wih = jnp.concatenate([Wih_f.T, Wih_b.T], axis=1).astype(bf16)
    bg = jnp.concatenate([(bih_f + bhh_f), (bih_b + bhh_b)]
                         ).reshape(1, 8 * HID2).astype(f32)
    woutp = jnp.zeros((HID, 128), f32).at[:, :NT].set(Wout.T.astype(f32))
    wcat = jnp.stack([
        jnp.concatenate([Whh_f.T.astype(f32), woutp[:HID2]], axis=1),
        jnp.concatenate([Whh_b.T.astype(f32), woutp[HID2:]], axis=1),
    ]).astype(bf16)

    plog = pl.pallas_call(
        _lstm_kernel,
        grid=(2,),
        in_specs=[
            pl.BlockSpec((L, _BHALF, D_MODEL), lambda i: (0, i, 0)),
            pl.BlockSpec((D_MODEL, 8 * HID2), lambda i: (0, 0)),
            pl.BlockSpec((2, HID2, 640), lambda i: (0, 0, 0)),
            pl.BlockSpec((1, 8 * HID2), lambda i: (0, 0)),
            pl.BlockSpec((2, _BHALF, HID2), lambda i: (0, i, 0)),
            pl.BlockSpec((2, _BHALF, HID2), lambda i: (0, i, 0)),
        ],
        out_specs=pl.BlockSpec((2, L, _BHALF, 128), lambda i: (0, 0, i, 0)),
        out_shape=jax.ShapeDtypeStruct((2, L, B, 128), bf16),
        scratch_shapes=[pltpu.VMEM((L, _BHALF, 8 * HID2), bf16)],
        compiler_params=pltpu.CompilerParams(
            dimension_semantics=("parallel",),
            vmem_limit_bytes=50 * 1024 * 1024),
        name="bilstm",
    )(x_tm, wih, wcat, bg, h0.astype(f32), c0.astype(f32))

    trans_pad = jnp.full((128, 128), NEG, f32).at[:NT, :NT].set(
        transitions.astype(f32))
    tcs = jnp.full((1, 128), NEG, f32).at[0, :NT].set(
        transitions[:, STOP].astype(f32))
    bout_r = jnp.zeros((1, 128), f32).at[0, :NT].set(bout.astype(f32))
    tagc = tags.T.astype(jnp.int32)[:, :, None]
    lab_prev = jnp.concatenate(
        [jnp.full((B, 1), START, tags.dtype), tags[:, :-1]], axis=1)
    prevc = lab_prev.T.astype(jnp.int32)[:, :, None]
    len_c = lengths.astype(jnp.int32)[:, None]

    parts = pl.pallas_call(
        _crf_kernel,
        grid=(2,),
        in_specs=[
            pl.BlockSpec((2, L, _BHALF, 128), lambda i: (0, 0, i, 0)),
            pl.BlockSpec((L, _BHALF, 1), lambda i: (0, i, 0)),
            pl.BlockSpec((L, _BHALF, 1), lambda i: (0, i, 0)),
            pl.BlockSpec((_BHALF, 1), lambda i: (i, 0)),
            pl.BlockSpec((128, 128), lambda i: (0, 0)),
            pl.BlockSpec((1, 128), lambda i: (0, 0)),
            pl.BlockSpec((1, 128), lambda i: (0, 0)),
        ],
        out_specs=pl.BlockSpec((1, 1, 128), lambda i: (i, 0, 0)),
        out_shape=jax.ShapeDtypeStruct((2, 1, 128), f32),
        scratch_shapes=[pltpu.VMEM((L, _BHALF, 128), f32),
                        pltpu.VMEM((L, _BHALF, 1), f32)],
        compiler_params=pltpu.CompilerParams(
            dimension_semantics=("parallel",),
            vmem_limit_bytes=50 * 1024 * 1024),
        name="crf_nll",
    )(plog, tagc, prevc, len_c, trans_pad, tcs, bout_r)

    return parts[0, 0, 0] + parts[1, 0, 0]


# speedup vs baseline: 3.2262x; 1.1556x over previous
import functools

import jax
import jax.numpy as jnp
from jax.experimental import pallas as pl
from jax.experimental.pallas import tpu as pltpu

B, L, V = 64, 256, 8000
D_EMB = 100
D_MODEL = 256
H, DK = 4, 64
HID, HID2 = 256, 128
NT = 20
START, STOP = 18, 19
NEG = -1.0e30

_BBA = 8
_BHALF = B // 2


def _attn_kernel(fin_ref, wq_ref, wk_ref, wv_ref, wo_ref,
                 bq_ref, bk_ref, bv_ref, bo_ref, x_ref):
    wq = wq_ref[...]
    wk = wk_ref[...]
    wv = wv_ref[...]
    wo = wo_ref[...]
    for ii in range(_BBA):
        f = fin_ref[ii].astype(jnp.bfloat16)
        q = jnp.dot(f, wq, preferred_element_type=jnp.float32) + bq_ref[...]
        k = jnp.dot(f, wk, preferred_element_type=jnp.float32) + bk_ref[...]
        v = jnp.dot(f, wv, preferred_element_type=jnp.float32) + bv_ref[...]
        qb = q.astype(jnp.bfloat16)
        kb = k.astype(jnp.bfloat16)
        vb = v.astype(jnp.bfloat16)
        outs = []
        for h in range(H):
            sl = slice(h * DK, (h + 1) * DK)
            s = jax.lax.dot_general(
                qb[:, sl], kb[:, sl], (((1,), (1,)), ((), ())),
                preferred_element_type=jnp.float32) * 0.125
            m = jnp.max(s, axis=1, keepdims=True)
            e = jnp.exp(s - m)
            l = jnp.sum(e, axis=1, keepdims=True)
            o = jnp.dot(e.astype(jnp.bfloat16), vb[:, sl],
                        preferred_element_type=jnp.float32)
            outs.append(o / l)
        cat = jnp.concatenate(outs, axis=1)
        xo = jnp.dot(cat.astype(jnp.bfloat16), wo,
                     preferred_element_type=jnp.float32) + bo_ref[...]
        x_ref[ii] = xo.astype(jnp.bfloat16)


def _lstm_kernel(x_ref, wih_ref, wcat_ref, bg_ref, h0_ref, c0_ref,
                 plog_ref, xg_ref):
    for half in range(2):
        xall = x_ref[:, half * _BHALF:(half + 1) * _BHALF, :].reshape(
            L * _BHALF, D_MODEL)
        xg = jnp.dot(xall, wih_ref[...], preferred_element_type=jnp.float32)
        xg_ref[:, half * _BHALF:(half + 1) * _BHALF, :] = (
            xg.astype(jnp.bfloat16).reshape(L, _BHALF, 8 * HID2))

    wf = wcat_ref[0]
    wb = wcat_ref[1]
    G4 = 4 * HID2
    bgf = bg_ref[...][:, :G4]
    bgb = bg_ref[...][:, G4:]

    def sig(z):
        return 0.5 * jnp.tanh(0.5 * z) + 0.5

    def step(t, carry):
        hf, cf, hb, cb = carry
        tb = (L - 1) - t
        yf = jnp.dot(hf.astype(jnp.bfloat16), wf,
                     preferred_element_type=jnp.float32)
        yb = jnp.dot(hb.astype(jnp.bfloat16), wb,
                     preferred_element_type=jnp.float32)
        plog_ref[0, jnp.maximum(t - 1, 0)] = yf[:, G4:].astype(jnp.bfloat16)
        plog_ref[1, jnp.minimum(tb + 1, L - 1)] = yb[:, G4:].astype(
            jnp.bfloat16)
        gf = (xg_ref[t][:, :G4].astype(jnp.float32) + bgf) + yf[:, :G4]
        gb = (xg_ref[tb][:, G4:].astype(jnp.float32) + bgb) + yb[:, :G4]
        i_f = sig(gf[:, 0:HID2])
        f_f = sig(gf[:, HID2:2 * HID2])
        g_f = jnp.tanh(gf[:, 2 * HID2:3 * HID2])
        o_f = sig(gf[:, 3 * HID2:])
        cf = f_f * cf + i_f * g_f
        hf = o_f * jnp.tanh(cf)
        i_b = sig(gb[:, 0:HID2])
        f_b = sig(gb[:, HID2:2 * HID2])
        g_b = jnp.tanh(gb[:, 2 * HID2:3 * HID2])
        o_b = sig(gb[:, 3 * HID2:])
        cb = f_b * cb + i_b * g_b
        hb = o_b * jnp.tanh(cb)
        return hf, cf, hb, cb

    init = (h0_ref[0], c0_ref[0], h0_ref[1], c0_ref[1])
    hf, cf, hb, cb = jax.lax.fori_loop(0, L, step, init, unroll=4)
    plog_ref[0, L - 1] = jnp.dot(
        hf.astype(jnp.bfloat16), wf[:, G4:],
        preferred_element_type=jnp.float32).astype(jnp.bfloat16)
    plog_ref[1, 0] = jnp.dot(
        hb.astype(jnp.bfloat16), wb[:, G4:],
        preferred_element_type=jnp.float32).astype(jnp.bfloat16)


def _crf_kernel(plog_ref, tag_ref, prev_ref, len_ref, trans_ref, tcs_ref,
                bout_ref, out_ref, logits_ref):
    logits_ref[...] = (plog_ref[0].astype(jnp.float32)
                       + plog_ref[1].astype(jnp.float32) + bout_ref[...])

    lane3 = jax.lax.broadcasted_iota(jnp.int32, (L, B, 128), 2)
    t3 = jax.lax.broadcasted_iota(jnp.int32, (L, B, 128), 0)
    tag3 = jnp.broadcast_to(tag_ref[...].astype(jnp.int32), (L, B, 128))
    prev3 = jnp.broadcast_to(prev_ref[...].astype(jnp.int32), (L, B, 128))
    len3 = jnp.broadcast_to(len_ref[...][None, :, :], (L, B, 128))

    curoh = jnp.where(lane3 == tag3, 1.0, 0.0)
    prevoh = jnp.where(lane3 == prev3, 1.0, 0.0)
    maskf = jnp.where(t3 < len3, 1.0, 0.0)

    transb = trans_ref[...].astype(jnp.bfloat16)
    rowvals = jnp.dot(
        prevoh.astype(jnp.bfloat16).reshape(L * B, 128), transb,
        preferred_element_type=jnp.float32).reshape(L, B, 128)
    lastsel = jnp.where(t3 == (len3 - 1), 1.0, 0.0)
    acc3 = ((logits_ref[...] + rowvals) * curoh * maskf
            + lastsel * curoh * tcs_ref[...][None, :, :])
    realb = jnp.sum(jnp.sum(acc3, axis=2, keepdims=True), axis=0)

    eb = jnp.exp(trans_ref[...]).astype(jnp.bfloat16)
    lane2 = jax.lax.broadcasted_iota(jnp.int32, (B, 128), 1)
    prev0 = jnp.where(lane2 < NT, 0.0, NEG)
    lenb = len_ref[...]

    tmx = jnp.max(jnp.max(trans_ref[...], axis=1, keepdims=True),
                  axis=0, keepdims=True)
    gm = jnp.max(logits_ref[...], axis=2, keepdims=True)
    gs = jnp.maximum(gm + (tmx + 2.995732273553991)[None, :, :], 0.0)
    gprev = jnp.concatenate(
        [jnp.zeros((1, B, 1), jnp.float32), gs[:-1]], axis=0)
    logits_ref[:, :, 127:128] = gs + gprev

    def step(t, carry):
        prv, m1, m0 = carry
        lg = logits_ref[t]
        mu = m0 + lg[:, 127:128]
        p = jnp.exp(prv - mu).astype(jnp.bfloat16)
        s = jnp.dot(p, eb, preferred_element_type=jnp.float32)
        new = jnp.maximum(jnp.log(s), NEG) + mu + lg
        prvn = jnp.where(t < lenb, new, prv)
        m1n = jnp.max(prvn, axis=1, keepdims=True)
        return prvn, m1n, m1

    m_init = jnp.zeros((B, 1), jnp.float32)
    prv, _, _ = jax.lax.fori_loop(0, L, step, (prev0, m_init, m_init),
                                  unroll=4)
    v = prv + tcs_ref[...]
    m2 = jnp.max(v, axis=1, keepdims=True)
    tot = jnp.log(jnp.sum(jnp.exp(v - m2), axis=1, keepdims=True)) + m2
    part = jnp.sum(tot - realb)
    out_ref[...] = jnp.zeros((1, 1, 128), jnp.float32) + part


def kernel(sentences, bushou, pinyin, weizhi, trans_weizhi, tags, lengths,
           emb1, emb2, Wq, bq, Wk, bk, Wv, bv, Wo, bo,
           Wih_f, Whh_f, bih_f, bhh_f, Wih_b, Whh_b, bih_b, bhh_b,
           Wout, bout, transitions, h0, c0):
    f32 = jnp.float32
    bf16 = jnp.bfloat16
    e1 = emb1[sentences]
    e2 = emb2[bushou]
    fin = jnp.concatenate([e1, e2, pinyin, trans_weizhi, weizhi],
                          axis=2).astype(f32)

    wq = Wq.T.astype(bf16)
    wk = Wk.T.astype(bf16)
    wv = Wv.T.astype(bf16)
    wo = Wo.T.astype(bf16)
    bq_r = bq.reshape(1, D_MODEL).astype(f32)
    bk_r = bk.reshape(1, D_MODEL).astype(f32)
    bv_r = bv.reshape(1, D_MODEL).astype(f32)
    bo_r = bo.reshape(1, D_MODEL).astype(f32)

    nblk = B // _BBA
    x_bm = pl.pallas_call(
        _attn_kernel,
        grid=(2, nblk // 2),
        in_specs=[
            pl.BlockSpec((_BBA, L, D_MODEL),
                         lambda c, i: (c * (nblk // 2) + i, 0, 0)),
            pl.BlockSpec((D_MODEL, D_MODEL), lambda c, i: (0, 0)),
            pl.BlockSpec((D_MODEL, D_MODEL), lambda c, i: (0, 0)),
            pl.BlockSpec((D_MODEL, D_MODEL), lambda c, i: (0, 0)),
            pl.BlockSpec((D_MODEL, D_MODEL), lambda c, i: (0, 0)),
            pl.BlockSpec((1, D_MODEL), lambda c, i: (0, 0)),
            pl.BlockSpec((1, D_MODEL), lambda c, i: (0, 0)),
            pl.BlockSpec((1, D_MODEL), lambda c, i: (0, 0)),
            pl.BlockSpec((1, D_MODEL), lambda c, i: (0, 0)),
        ],
        out_specs=pl.BlockSpec((_BBA, L, D_MODEL),
                               lambda c, i: (c * (nblk // 2) + i, 0, 0)),
        out_shape=jax.ShapeDtypeStruct((B, L, D_MODEL), bf16),
        compiler_params=pltpu.CompilerParams(
            dimension_semantics=("parallel", "parallel")),
        name="attn_fused",
    )(fin, wq, wk, wv, wo, bq_r, bk_r, bv_r, bo_r)

    x_tm = x_bm.transpose(1, 0, 2)

    wih = jnp.concatenate([Wih_f.T, Wih_b.T], axis=1).astype(bf16)
    bg = jnp.concatenate([(bih_f + bhh_f), (bih_b + bhh_b)]
                         ).reshape(1, 8 * HID2).astype(f32)
    woutp = jnp.zeros((HID, 128), f32).at[:, :NT].set(Wout.T.astype(f32))
    wcat = jnp.stack([
        jnp.concatenate([Whh_f.T.astype(f32), woutp[:HID2]], axis=1),
        jnp.concatenate([Whh_b.T.astype(f32), woutp[HID2:]], axis=1),
    ]).astype(bf16)

    plog = pl.pallas_call(
        _lstm_kernel,
        grid=(1,),
        in_specs=[
            pl.BlockSpec((L, B, D_MODEL), lambda i: (0, 0, 0)),
            pl.BlockSpec((D_MODEL, 8 * HID2), lambda i: (0, 0)),
            pl.BlockSpec((2, HID2, 640), lambda i: (0, 0, 0)),
            pl.BlockSpec((1, 8 * HID2), lambda i: (0, 0)),
            pl.BlockSpec((2, B, HID2), lambda i: (0, 0, 0)),
            pl.BlockSpec((2, B, HID2), lambda i: (0, 0, 0)),
        ],
        out_specs=pl.BlockSpec((2, L, B, 128), lambda i: (0, 0, 0, 0)),
        out_shape=jax.ShapeDtypeStruct((2, L, B, 128), bf16),
        scratch_shapes=[pltpu.VMEM((L, B, 8 * HID2), bf16)],
        compiler_params=pltpu.CompilerParams(
            dimension_semantics=("arbitrary",),
            vmem_limit_bytes=56 * 1024 * 1024),
        name="bilstm",
    )(x_tm, wih, wcat, bg, h0.astype(f32), c0.astype(f32))

    trans_pad = jnp.full((128, 128), NEG, f32).at[:NT, :NT].set(
        transitions.astype(f32))
    tcs = jnp.full((1, 128), NEG, f32).at[0, :NT].set(
        transitions[:, STOP].astype(f32))
    bout_r = jnp.zeros((1, 128), f32).at[0, :NT].set(bout.astype(f32))
    tagc = tags.T.astype(jnp.int8)[:, :, None]
    lab_prev = jnp.concatenate(
        [jnp.full((B, 1), START, tags.dtype), tags[:, :-1]], axis=1)
    prevc = lab_prev.T.astype(jnp.int8)[:, :, None]
    len_c = lengths.astype(jnp.int32)[:, None]

    parts = pl.pallas_call(
        _crf_kernel,
        grid=(1,),
        in_specs=[
            pl.BlockSpec((2, L, B, 128), lambda i: (0, 0, 0, 0)),
            pl.BlockSpec((L, B, 1), lambda i: (0, 0, 0)),
            pl.BlockSpec((L, B, 1), lambda i: (0, 0, 0)),
            pl.BlockSpec((B, 1), lambda i: (0, 0)),
            pl.BlockSpec((128, 128), lambda i: (0, 0)),
            pl.BlockSpec((1, 128), lambda i: (0, 0)),
            pl.BlockSpec((1, 128), lambda i: (0, 0)),
        ],
        out_specs=pl.BlockSpec((1, 1, 128), lambda i: (0, 0, 0)),
        out_shape=jax.ShapeDtypeStruct((1, 1, 128), f32),
        scratch_shapes=[pltpu.VMEM((L, B, 128), f32)],
        compiler_params=pltpu.CompilerParams(
            dimension_semantics=("arbitrary",),
            vmem_limit_bytes=56 * 1024 * 1024),
        name="crf_nll",
    )(plog, tagc, prevc, len_c, trans_pad, tcs, bout_r)

    return parts[0, 0, 0]


# speedup vs baseline: 3.9260x; 1.2169x over previous
import functools

import jax
import jax.numpy as jnp
from jax.experimental import pallas as pl
from jax.experimental.pallas import tpu as pltpu

B, L, V = 64, 256, 8000
D_EMB = 100
D_MODEL = 256
H, DK = 4, 64
HID, HID2 = 256, 128
NT = 20
START, STOP = 18, 19
NEG = -1.0e30

_BBA = 8
_BHALF = B // 2


def _attn_kernel(sent_ref, bush_ref, e1p_ref, e2a_ref, e2b_ref, rest_ref,
                 wq_ref, wk_ref, wv_ref, wo_ref,
                 bq_ref, bk_ref, bv_ref, bo_ref, x_ref, fin_ref):
    wq = wq_ref[...]
    wk = wk_ref[...]
    wv = wv_ref[...]
    wo = wo_ref[...]
    blk = pl.program_id(0) * (B // _BBA // 2) + pl.program_id(1)
    for ii in range(_BBA):
        item = blk * _BBA + ii
        for g in range(L // 8):
            rows0 = []
            rows1 = []
            for j in range(8):
                t = g * 8 + j
                tok1 = sent_ref[item, t]
                tok2 = bush_ref[item, t]
                rows0.append(e1p_ref[pl.ds(tok1, 1), :]
                             + e2a_ref[pl.ds(tok2, 1), :])
                rows1.append(e2b_ref[pl.ds(tok2, 1), :])
            lo = jnp.concatenate(rows0, axis=0)
            hi = (jnp.concatenate(rows1, axis=0)
                  + rest_ref[ii, g * 8:(g + 1) * 8, :])
            fin_ref[ii, g * 8:(g + 1) * 8, 0:128] = lo.astype(jnp.bfloat16)
            fin_ref[ii, g * 8:(g + 1) * 8, 128:256] = hi.astype(jnp.bfloat16)
        f = fin_ref[ii]
        q = jnp.dot(f, wq, preferred_element_type=jnp.float32) + bq_ref[...]
        k = jnp.dot(f, wk, preferred_element_type=jnp.float32) + bk_ref[...]
        v = jnp.dot(f, wv, preferred_element_type=jnp.float32) + bv_ref[...]
        qb = q.astype(jnp.bfloat16)
        kb = k.astype(jnp.bfloat16)
        vb = v.astype(jnp.bfloat16)
        outs = []
        for h in range(H):
            sl = slice(h * DK, (h + 1) * DK)
            s = jax.lax.dot_general(
                qb[:, sl], kb[:, sl], (((1,), (1,)), ((), ())),
                preferred_element_type=jnp.float32) * 0.125
            m = jnp.max(s, axis=1, keepdims=True)
            e = jnp.exp(s - m)
            l = jnp.sum(e, axis=1, keepdims=True)
            o = jnp.dot(e.astype(jnp.bfloat16), vb[:, sl],
                        preferred_element_type=jnp.float32)
            outs.append(o / l)
        cat = jnp.concatenate(outs, axis=1)
        xo = jnp.dot(cat.astype(jnp.bfloat16), wo,
                     preferred_element_type=jnp.float32) + bo_ref[...]
        x_ref[ii] = xo.astype(jnp.bfloat16)


def _lstm_kernel(x_ref, wih_ref, wcat_ref, bg_ref, h0_ref, c0_ref,
                 plog_ref, xg_ref):
    for half in range(2):
        xall = x_ref[:, half * _BHALF:(half + 1) * _BHALF, :].reshape(
            L * _BHALF, D_MODEL)
        xg = jnp.dot(xall, wih_ref[...], preferred_element_type=jnp.float32)
        xg_ref[:, half * _BHALF:(half + 1) * _BHALF, :] = (
            xg.astype(jnp.bfloat16).reshape(L, _BHALF, 8 * HID2))

    wf = wcat_ref[0]
    wb = wcat_ref[1]
    G4 = 4 * HID2
    bgf = bg_ref[...][:, :G4]
    bgb = bg_ref[...][:, G4:]

    def sig(z):
        return 0.5 * jnp.tanh(0.5 * z) + 0.5

    def step(t, carry):
        hf, cf, hb, cb = carry
        tb = (L - 1) - t
        yf = jnp.dot(hf.astype(jnp.bfloat16), wf,
                     preferred_element_type=jnp.float32)
        yb = jnp.dot(hb.astype(jnp.bfloat16), wb,
                     preferred_element_type=jnp.float32)
        plog_ref[0, jnp.maximum(t - 1, 0)] = yf[:, G4:].astype(jnp.bfloat16)
        plog_ref[1, jnp.minimum(tb + 1, L - 1)] = yb[:, G4:].astype(
            jnp.bfloat16)
        gf = (xg_ref[t][:, :G4].astype(jnp.float32) + bgf) + yf[:, :G4]
        gb = (xg_ref[tb][:, G4:].astype(jnp.float32) + bgb) + yb[:, :G4]
        i_f = sig(gf[:, 0:HID2])
        f_f = sig(gf[:, HID2:2 * HID2])
        g_f = jnp.tanh(gf[:, 2 * HID2:3 * HID2])
        o_f = sig(gf[:, 3 * HID2:])
        cf = f_f * cf + i_f * g_f
        hf = o_f * jnp.tanh(cf)
        i_b = sig(gb[:, 0:HID2])
        f_b = sig(gb[:, HID2:2 * HID2])
        g_b = jnp.tanh(gb[:, 2 * HID2:3 * HID2])
        o_b = sig(gb[:, 3 * HID2:])
        cb = f_b * cb + i_b * g_b
        hb = o_b * jnp.tanh(cb)
        return hf, cf, hb, cb

    init = (h0_ref[0], c0_ref[0], h0_ref[1], c0_ref[1])
    hf, cf, hb, cb = jax.lax.fori_loop(0, L, step, init, unroll=4)
    plog_ref[0, L - 1] = jnp.dot(
        hf.astype(jnp.bfloat16), wf[:, G4:],
        preferred_element_type=jnp.float32).astype(jnp.bfloat16)
    plog_ref[1, 0] = jnp.dot(
        hb.astype(jnp.bfloat16), wb[:, G4:],
        preferred_element_type=jnp.float32).astype(jnp.bfloat16)


def _crf_kernel(plog_ref, tag_ref, prev_ref, len_ref, trans_ref, tcs_ref,
                bout_ref, out_ref, logits_ref):
    logits_ref[...] = (plog_ref[0].astype(jnp.float32)
                       + plog_ref[1].astype(jnp.float32) + bout_ref[...])

    lane3 = jax.lax.broadcasted_iota(jnp.int32, (L, B, 128), 2)
    t3 = jax.lax.broadcasted_iota(jnp.int32, (L, B, 128), 0)
    tag3 = jnp.broadcast_to(tag_ref[...].astype(jnp.int32), (L, B, 128))
    prev3 = jnp.broadcast_to(prev_ref[...].astype(jnp.int32), (L, B, 128))
    len3 = jnp.broadcast_to(len_ref[...][None, :, :], (L, B, 128))

    curoh = jnp.where(lane3 == tag3, 1.0, 0.0)
    prevoh = jnp.where(lane3 == prev3, 1.0, 0.0)
    maskf = jnp.where(t3 < len3, 1.0, 0.0)

    transb = trans_ref[...].astype(jnp.bfloat16)
    rowvals = jnp.dot(
        prevoh.astype(jnp.bfloat16).reshape(L * B, 128), transb,
        preferred_element_type=jnp.float32).reshape(L, B, 128)
    lastsel = jnp.where(t3 == (len3 - 1), 1.0, 0.0)
    acc3 = ((logits_ref[...] + rowvals) * curoh * maskf
            + lastsel * curoh * tcs_ref[...][None, :, :])
    realb = jnp.sum(jnp.sum(acc3, axis=2, keepdims=True), axis=0)

    eb = jnp.exp(trans_ref[...]).astype(jnp.bfloat16)
    lane2 = jax.lax.broadcasted_iota(jnp.int32, (B, 128), 1)
    prev0 = jnp.where(lane2 < NT, 0.0, NEG)
    lenb = len_ref[...]

    tmx = jnp.max(jnp.max(trans_ref[...], axis=1, keepdims=True),
                  axis=0, keepdims=True)
    gm = jnp.max(logits_ref[...], axis=2, keepdims=True)
    gs = jnp.maximum(gm + (tmx + 2.995732273553991)[None, :, :], 0.0)
    gprev = jnp.concatenate(
        [jnp.zeros((1, B, 1), jnp.float32), gs[:-1]], axis=0)
    logits_ref[:, :, 127:128] = gs + gprev

    def step(t, carry):
        prv, m1, m0 = carry
        lg = logits_ref[t]
        mu = m0 + lg[:, 127:128]
        p = jnp.exp(prv - mu).astype(jnp.bfloat16)
        s = jnp.dot(p, eb, preferred_element_type=jnp.float32)
        new = jnp.maximum(jnp.log(s), NEG) + mu + lg
        prvn = jnp.where(t < lenb, new, prv)
        m1n = jnp.max(prvn, axis=1, keepdims=True)
        return prvn, m1n, m1

    m_init = jnp.zeros((B, 1), jnp.float32)
    prv, _, _ = jax.lax.fori_loop(0, L, step, (prev0, m_init, m_init),
                                  unroll=4)
    v = prv + tcs_ref[...]
    m2 = jnp.max(v, axis=1, keepdims=True)
    tot = jnp.log(jnp.sum(jnp.exp(v - m2), axis=1, keepdims=True)) + m2
    part = jnp.sum(tot - realb)
    out_ref[...] = jnp.zeros((1, 1, 128), jnp.float32) + part


def kernel(sentences, bushou, pinyin, weizhi, trans_weizhi, tags, lengths,
           emb1, emb2, Wq, bq, Wk, bk, Wv, bv, Wo, bo,
           Wih_f, Whh_f, bih_f, bhh_f, Wih_b, Whh_b, bih_b, bhh_b,
           Wout, bout, transitions, h0, c0):
    f32 = jnp.float32
    bf16 = jnp.bfloat16
    e1p = jnp.zeros((V, 128), f32).at[:, :D_EMB].set(emb1.astype(f32))
    e2a = jnp.zeros((V, 128), f32).at[:, D_EMB:].set(
        emb2[:, :128 - D_EMB].astype(f32))
    e2b = jnp.zeros((V, 128), f32).at[:, :2 * D_EMB - 128].set(
        emb2[:, 128 - D_EMB:].astype(f32))
    rest = jnp.zeros((B, L, 128), f32).at[:, :, 2 * D_EMB - 128:].set(
        jnp.concatenate([pinyin, trans_weizhi, weizhi], axis=2).astype(f32))
    sent_i = sentences.astype(jnp.int32)
    bush_i = bushou.astype(jnp.int32)

    wq = Wq.T.astype(bf16)
    wk = Wk.T.astype(bf16)
    wv = Wv.T.astype(bf16)
    wo = Wo.T.astype(bf16)
    bq_r = bq.reshape(1, D_MODEL).astype(f32)
    bk_r = bk.reshape(1, D_MODEL).astype(f32)
    bv_r = bv.reshape(1, D_MODEL).astype(f32)
    bo_r = bo.reshape(1, D_MODEL).astype(f32)

    nblk = B // _BBA
    x_bm = pl.pallas_call(
        _attn_kernel,
        grid=(2, nblk // 2),
        in_specs=[
            pl.BlockSpec(memory_space=pltpu.SMEM),
            pl.BlockSpec(memory_space=pltpu.SMEM),
            pl.BlockSpec((V, 128), lambda c, i: (0, 0)),
            pl.BlockSpec((V, 128), lambda c, i: (0, 0)),
            pl.BlockSpec((V, 128), lambda c, i: (0, 0)),
            pl.BlockSpec((_BBA, L, 128),
                         lambda c, i: (c * (nblk // 2) + i, 0, 0)),
            pl.BlockSpec((D_MODEL, D_MODEL), lambda c, i: (0, 0)),
            pl.BlockSpec((D_MODEL, D_MODEL), lambda c, i: (0, 0)),
            pl.BlockSpec((D_MODEL, D_MODEL), lambda c, i: (0, 0)),
            pl.BlockSpec((D_MODEL, D_MODEL), lambda c, i: (0, 0)),
            pl.BlockSpec((1, D_MODEL), lambda c, i: (0, 0)),
            pl.BlockSpec((1, D_MODEL), lambda c, i: (0, 0)),
            pl.BlockSpec((1, D_MODEL), lambda c, i: (0, 0)),
            pl.BlockSpec((1, D_MODEL), lambda c, i: (0, 0)),
        ],
        out_specs=pl.BlockSpec((_BBA, L, D_MODEL),
                               lambda c, i: (c * (nblk // 2) + i, 0, 0)),
        out_shape=jax.ShapeDtypeStruct((B, L, D_MODEL), bf16),
        scratch_shapes=[pltpu.VMEM((_BBA, L, D_MODEL), bf16)],
        compiler_params=pltpu.CompilerParams(
            dimension_semantics=("parallel", "parallel"),
            vmem_limit_bytes=48 * 1024 * 1024),
        name="attn_fused",
    )(sent_i, bush_i, e1p, e2a, e2b, rest,
      wq, wk, wv, wo, bq_r, bk_r, bv_r, bo_r)

    x_tm = x_bm.transpose(1, 0, 2)

    wih = jnp.concatenate([Wih_f.T, Wih_b.T], axis=1).astype(bf16)
    bg = jnp.concatenate([(bih_f + bhh_f), (bih_b + bhh_b)]
                         ).reshape(1, 8 * HID2).astype(f32)
    woutp = jnp.zeros((HID, 128), f32).at[:, :NT].set(Wout.T.astype(f32))
    wcat = jnp.stack([
        jnp.concatenate([Whh_f.T.astype(f32), woutp[:HID2]], axis=1),
        jnp.concatenate([Whh_b.T.astype(f32), woutp[HID2:]], axis=1),
    ]).astype(bf16)

    plog = pl.pallas_call(
        _lstm_kernel,
        grid=(1,),
        in_specs=[
            pl.BlockSpec((L, B, D_MODEL), lambda i: (0, 0, 0)),
            pl.BlockSpec((D_MODEL, 8 * HID2), lambda i: (0, 0)),
            pl.BlockSpec((2, HID2, 640), lambda i: (0, 0, 0)),
            pl.BlockSpec((1, 8 * HID2), lambda i: (0, 0)),
            pl.BlockSpec((2, B, HID2), lambda i: (0, 0, 0)),
            pl.BlockSpec((2, B, HID2), lambda i: (0, 0, 0)),
        ],
        out_specs=pl.BlockSpec((2, L, B, 128), lambda i: (0, 0, 0, 0)),
        out_shape=jax.ShapeDtypeStruct((2, L, B, 128), bf16),
        scratch_shapes=[pltpu.VMEM((L, B, 8 * HID2), bf16)],
        compiler_params=pltpu.CompilerParams(
            dimension_semantics=("arbitrary",),
            vmem_limit_bytes=56 * 1024 * 1024),
        name="bilstm",
    )(x_tm, wih, wcat, bg, h0.astype(f32), c0.astype(f32))

    trans_pad = jnp.full((128, 128), NEG, f32).at[:NT, :NT].set(
        transitions.astype(f32))
    tcs = jnp.full((1, 128), NEG, f32).at[0, :NT].set(
        transitions[:, STOP].astype(f32))
    bout_r = jnp.zeros((1, 128), f32).at[0, :NT].set(bout.astype(f32))
    tagc = tags.T.astype(jnp.int8)[:, :, None]
    lab_prev = jnp.concatenate(
        [jnp.full((B, 1), START, tags.dtype), tags[:, :-1]], axis=1)
    prevc = lab_prev.T.astype(jnp.int8)[:, :, None]
    len_c = lengths.astype(jnp.int32)[:, None]

    parts = pl.pallas_call(
        _crf_kernel,
        grid=(1,),
        in_specs=[
            pl.BlockSpec((2, L, B, 128), lambda i: (0, 0, 0, 0)),
            pl.BlockSpec((L, B, 1), lambda i: (0, 0, 0)),
            pl.BlockSpec((L, B, 1), lambda i: (0, 0, 0)),
            pl.BlockSpec((B, 1), lambda i: (0, 0)),
            pl.BlockSpec((128, 128), lambda i: (0, 0)),
            pl.BlockSpec((1, 128), lambda i: (0, 0)),
            pl.BlockSpec((1, 128), lambda i: (0, 0)),
        ],
        out_specs=pl.BlockSpec((1, 1, 128), lambda i: (0, 0, 0)),
        out_shape=jax.ShapeDtypeStruct((1, 1, 128), f32),
        scratch_shapes=[pltpu.VMEM((L, B, 128), f32)],
        compiler_params=pltpu.CompilerParams(
            dimension_semantics=("arbitrary",),
            vmem_limit_bytes=56 * 1024 * 1024),
        name="crf_nll",
    )(plog, tagc, prevc, len_c, trans_pad, tcs, bout_r)

    return parts[0, 0, 0]


# speedup vs baseline: 3.9890x; 1.0161x over previous
import functools

import jax
import jax.numpy as jnp
from jax.experimental import pallas as pl
from jax.experimental.pallas import tpu as pltpu

B, L, V = 64, 256, 8000
D_EMB = 100
D_MODEL = 256
H, DK = 4, 64
HID, HID2 = 256, 128
NT = 20
START, STOP = 18, 19
NEG = -1.0e30
_DNT = (((1,), (1,)), ((), ()))

_BBA = 8
_BHALF = B // 2


def _attn_kernel(sent_ref, bush_ref, e1p_ref, e2a_ref, e2b_ref, rest_ref,
                 wq_ref, wk_ref, wv_ref, wo_ref,
                 bq_ref, bk_ref, bv_ref, bo_ref, x_ref, fin_ref):
    wq = wq_ref[...]
    wk = wk_ref[...]
    wv = wv_ref[...]
    wo = wo_ref[...]
    blk = pl.program_id(0) * (B // _BBA // 2) + pl.program_id(1)
    for ii in range(_BBA):
        item = blk * _BBA + ii
        for g in range(L // 8):
            rows0 = []
            rows1 = []
            for j in range(8):
                t = g * 8 + j
                tok1 = sent_ref[item, t]
                tok2 = bush_ref[item, t]
                rows0.append(e1p_ref[pl.ds(tok1, 1), :]
                             + e2a_ref[pl.ds(tok2, 1), :])
                rows1.append(e2b_ref[pl.ds(tok2, 1), :])
            lo = jnp.concatenate(rows0, axis=0)
            hi = (jnp.concatenate(rows1, axis=0)
                  + rest_ref[ii, g * 8:(g + 1) * 8, :])
            fin_ref[ii, g * 8:(g + 1) * 8, 0:128] = lo.astype(jnp.bfloat16)
            fin_ref[ii, g * 8:(g + 1) * 8, 128:256] = hi.astype(jnp.bfloat16)
        f = fin_ref[ii]
        dn = (((1,), (1,)), ((), ()))
        q = jax.lax.dot_general(f, wq, dn,
                                preferred_element_type=jnp.float32) + bq_ref[...]
        k = jax.lax.dot_general(f, wk, dn,
                                preferred_element_type=jnp.float32) + bk_ref[...]
        v = jax.lax.dot_general(f, wv, dn,
                                preferred_element_type=jnp.float32) + bv_ref[...]
        qb = q.astype(jnp.bfloat16)
        kb = k.astype(jnp.bfloat16)
        vb = v.astype(jnp.bfloat16)
        outs = []
        for h in range(H):
            sl = slice(h * DK, (h + 1) * DK)
            s = jax.lax.dot_general(
                qb[:, sl], kb[:, sl], (((1,), (1,)), ((), ())),
                preferred_element_type=jnp.float32) * 0.125
            m = jnp.max(s, axis=1, keepdims=True)
            e = jnp.exp(s - m)
            l = jnp.sum(e, axis=1, keepdims=True)
            o = jnp.dot(e.astype(jnp.bfloat16), vb[:, sl],
                        preferred_element_type=jnp.float32)
            outs.append(o / l)
        cat = jnp.concatenate(outs, axis=1)
        xo = jax.lax.dot_general(cat.astype(jnp.bfloat16), wo, dn,
                                 preferred_element_type=jnp.float32) + bo_ref[...]
        x_ref[ii] = xo.astype(jnp.bfloat16)


def _lstm_kernel(x_ref, wih_ref, wcat_ref, bg_ref, h0_ref, c0_ref,
                 plog_ref, xg_ref):
    for half in range(2):
        xall = x_ref[:, half * _BHALF:(half + 1) * _BHALF, :].reshape(
            L * _BHALF, D_MODEL)
        xg = jax.lax.dot_general(xall, wih_ref[...], (((1,), (1,)), ((), ())),
                                 preferred_element_type=jnp.float32)
        xg_ref[:, half * _BHALF:(half + 1) * _BHALF, :] = (
            xg.astype(jnp.bfloat16).reshape(L, _BHALF, 8 * HID2))

    wf = wcat_ref[0]
    wb = wcat_ref[1]
    G4 = 4 * HID2
    bgf = bg_ref[...][:, :G4]
    bgb = bg_ref[...][:, G4:]

    def sig(z):
        return 0.5 * jnp.tanh(0.5 * z) + 0.5

    def step(t, carry):
        hf, cf, hb, cb = carry
        tb = (L - 1) - t
        yf = jax.lax.dot_general(hf.astype(jnp.bfloat16), wf, _DNT,
                                 preferred_element_type=jnp.float32)
        yb = jax.lax.dot_general(hb.astype(jnp.bfloat16), wb, _DNT,
                                 preferred_element_type=jnp.float32)
        plog_ref[0, jnp.maximum(t - 1, 0)] = yf[:, G4:].astype(jnp.bfloat16)
        plog_ref[1, jnp.minimum(tb + 1, L - 1)] = yb[:, G4:].astype(
            jnp.bfloat16)
        gf = (xg_ref[t][:, :G4].astype(jnp.float32) + bgf) + yf[:, :G4]
        gb = (xg_ref[tb][:, G4:].astype(jnp.float32) + bgb) + yb[:, :G4]
        i_f = sig(gf[:, 0:HID2])
        f_f = sig(gf[:, HID2:2 * HID2])
        g_f = jnp.tanh(gf[:, 2 * HID2:3 * HID2])
        o_f = sig(gf[:, 3 * HID2:])
        cf = f_f * cf + i_f * g_f
        hf = o_f * jnp.tanh(cf)
        i_b = sig(gb[:, 0:HID2])
        f_b = sig(gb[:, HID2:2 * HID2])
        g_b = jnp.tanh(gb[:, 2 * HID2:3 * HID2])
        o_b = sig(gb[:, 3 * HID2:])
        cb = f_b * cb + i_b * g_b
        hb = o_b * jnp.tanh(cb)
        return hf, cf, hb, cb

    init = (h0_ref[0], c0_ref[0], h0_ref[1], c0_ref[1])
    hf, cf, hb, cb = jax.lax.fori_loop(0, L, step, init, unroll=4)
    plog_ref[0, L - 1] = jax.lax.dot_general(
        hf.astype(jnp.bfloat16), wf[G4:, :], _DNT,
        preferred_element_type=jnp.float32).astype(jnp.bfloat16)
    plog_ref[1, 0] = jax.lax.dot_general(
        hb.astype(jnp.bfloat16), wb[G4:, :], _DNT,
        preferred_element_type=jnp.float32).astype(jnp.bfloat16)


def _crf_kernel(plog_ref, tp_ref, len_ref, trans_ref, tcs_ref,
                bout_ref, out_ref, logits_ref):
    logits_ref[...] = (plog_ref[0].astype(jnp.float32)
                       + plog_ref[1].astype(jnp.float32) + bout_ref[...])

    lane3 = jax.lax.broadcasted_iota(jnp.int32, (L, B, 128), 2)
    t3 = jax.lax.broadcasted_iota(jnp.int32, (L, B, 128), 0)
    tag3 = jnp.broadcast_to(tp_ref[0].astype(jnp.int32), (L, B, 128))
    prev3 = jnp.broadcast_to(tp_ref[1].astype(jnp.int32), (L, B, 128))
    len3 = jnp.broadcast_to(len_ref[...][None, :, :], (L, B, 128))

    curoh = jnp.where(lane3 == tag3, 1.0, 0.0)
    prevoh = jnp.where(lane3 == prev3, 1.0, 0.0)
    maskf = jnp.where(t3 < len3, 1.0, 0.0)

    transb = trans_ref[...].astype(jnp.bfloat16)
    rowvals = jnp.dot(
        prevoh.astype(jnp.bfloat16).reshape(L * B, 128), transb,
        preferred_element_type=jnp.float32).reshape(L, B, 128)
    lastsel = jnp.where(t3 == (len3 - 1), 1.0, 0.0)
    acc3 = ((logits_ref[...] + rowvals) * curoh * maskf
            + lastsel * curoh * tcs_ref[...][None, :, :])
    realb = jnp.sum(jnp.sum(acc3, axis=2, keepdims=True), axis=0)

    eb = jnp.exp(trans_ref[...]).astype(jnp.bfloat16)
    lane2 = jax.lax.broadcasted_iota(jnp.int32, (B, 128), 1)
    prev0 = jnp.where(lane2 < NT, 0.0, NEG)
    lenb = len_ref[...]

    tmx = jnp.max(jnp.max(trans_ref[...], axis=1, keepdims=True),
                  axis=0, keepdims=True)
    gm = jnp.max(logits_ref[...], axis=2, keepdims=True)
    gs = jnp.maximum(gm + (tmx + 2.995732273553991)[None, :, :], 0.0)
    gprev = jnp.concatenate(
        [jnp.zeros((1, B, 1), jnp.float32), gs[:-1]], axis=0)
    logits_ref[:, :, 127:128] = gs + gprev

    def step(t, carry):
        prv, m1, m0 = carry
        lg = logits_ref[t]
        mu = m0 + lg[:, 127:128]
        p = jnp.exp(prv - mu).astype(jnp.bfloat16)
        s = jnp.dot(p, eb, preferred_element_type=jnp.float32)
        new = jnp.maximum(jnp.log(s), NEG) + mu + lg
        prvn = jnp.where(t < lenb, new, prv)
        m1n = jnp.max(prvn, axis=1, keepdims=True)
        return prvn, m1n, m1

    m_init = jnp.zeros((B, 1), jnp.float32)
    prv, _, _ = jax.lax.fori_loop(0, L, step, (prev0, m_init, m_init),
                                  unroll=4)
    v = prv + tcs_ref[...]
    m2 = jnp.max(v, axis=1, keepdims=True)
    tot = jnp.log(jnp.sum(jnp.exp(v - m2), axis=1, keepdims=True)) + m2
    part = jnp.sum(tot - realb)
    out_ref[...] = jnp.zeros((1, 1, 128), jnp.float32) + part


def kernel(sentences, bushou, pinyin, weizhi, trans_weizhi, tags, lengths,
           emb1, emb2, Wq, bq, Wk, bk, Wv, bv, Wo, bo,
           Wih_f, Whh_f, bih_f, bhh_f, Wih_b, Whh_b, bih_b, bhh_b,
           Wout, bout, transitions, h0, c0):
    f32 = jnp.float32
    bf16 = jnp.bfloat16
    e1p = jnp.pad(emb1.astype(f32), ((0, 0), (0, 128 - D_EMB)))
    e2a = jnp.pad(emb2[:, :128 - D_EMB].astype(f32), ((0, 0), (D_EMB, 0)))
    e2b = jnp.pad(emb2[:, 128 - D_EMB:].astype(f32),
                  ((0, 0), (0, 256 - 2 * D_EMB)))
    rest = jnp.pad(
        jnp.concatenate([pinyin, trans_weizhi, weizhi], axis=2).astype(f32),
        ((0, 0), (0, 0), (2 * D_EMB - 128, 0)))
    sent_i = sentences.astype(jnp.int32)
    bush_i = bushou.astype(jnp.int32)

    wq = Wq.astype(bf16)
    wk = Wk.astype(bf16)
    wv = Wv.astype(bf16)
    wo = Wo.astype(bf16)
    bq_r = bq.reshape(1, D_MODEL).astype(f32)
    bk_r = bk.reshape(1, D_MODEL).astype(f32)
    bv_r = bv.reshape(1, D_MODEL).astype(f32)
    bo_r = bo.reshape(1, D_MODEL).astype(f32)

    nblk = B // _BBA
    x_bm = pl.pallas_call(
        _attn_kernel,
        grid=(2, nblk // 2),
        in_specs=[
            pl.BlockSpec(memory_space=pltpu.SMEM),
            pl.BlockSpec(memory_space=pltpu.SMEM),
            pl.BlockSpec((V, 128), lambda c, i: (0, 0)),
            pl.BlockSpec((V, 128), lambda c, i: (0, 0)),
            pl.BlockSpec((V, 128), lambda c, i: (0, 0)),
            pl.BlockSpec((_BBA, L, 128),
                         lambda c, i: (c * (nblk // 2) + i, 0, 0)),
            pl.BlockSpec((D_MODEL, D_MODEL), lambda c, i: (0, 0)),
            pl.BlockSpec((D_MODEL, D_MODEL), lambda c, i: (0, 0)),
            pl.BlockSpec((D_MODEL, D_MODEL), lambda c, i: (0, 0)),
            pl.BlockSpec((D_MODEL, D_MODEL), lambda c, i: (0, 0)),
            pl.BlockSpec((1, D_MODEL), lambda c, i: (0, 0)),
            pl.BlockSpec((1, D_MODEL), lambda c, i: (0, 0)),
            pl.BlockSpec((1, D_MODEL), lambda c, i: (0, 0)),
            pl.BlockSpec((1, D_MODEL), lambda c, i: (0, 0)),
        ],
        out_specs=pl.BlockSpec((_BBA, L, D_MODEL),
                               lambda c, i: (c * (nblk // 2) + i, 0, 0)),
        out_shape=jax.ShapeDtypeStruct((B, L, D_MODEL), bf16),
        scratch_shapes=[pltpu.VMEM((_BBA, L, D_MODEL), bf16)],
        compiler_params=pltpu.CompilerParams(
            dimension_semantics=("parallel", "parallel"),
            vmem_limit_bytes=48 * 1024 * 1024),
        name="attn_fused",
    )(sent_i, bush_i, e1p, e2a, e2b, rest,
      wq, wk, wv, wo, bq_r, bk_r, bv_r, bo_r)

    x_tm = x_bm.transpose(1, 0, 2)

    wih = jnp.concatenate([Wih_f, Wih_b], axis=0).astype(bf16)
    bg = jnp.concatenate([(bih_f + bhh_f), (bih_b + bhh_b)]
                         ).reshape(1, 8 * HID2).astype(f32)
    woutp = jnp.zeros((2, 128, HID2), f32).at[0, :NT].set(
        Wout[:, :HID2].astype(f32)).at[1, :NT].set(Wout[:, HID2:].astype(f32))
    wcat = jnp.stack([
        jnp.concatenate([Whh_f.astype(f32), woutp[0]], axis=0),
        jnp.concatenate([Whh_b.astype(f32), woutp[1]], axis=0),
    ]).astype(bf16)

    plog = pl.pallas_call(
        _lstm_kernel,
        grid=(1,),
        in_specs=[
            pl.BlockSpec((L, B, D_MODEL), lambda i: (0, 0, 0)),
            pl.BlockSpec((8 * HID2, D_MODEL), lambda i: (0, 0)),
            pl.BlockSpec((2, 640, HID2), lambda i: (0, 0, 0)),
            pl.BlockSpec((1, 8 * HID2), lambda i: (0, 0)),
            pl.BlockSpec((2, B, HID2), lambda i: (0, 0, 0)),
            pl.BlockSpec((2, B, HID2), lambda i: (0, 0, 0)),
        ],
        out_specs=pl.BlockSpec((2, L, B, 128), lambda i: (0, 0, 0, 0)),
        out_shape=jax.ShapeDtypeStruct((2, L, B, 128), bf16),
        scratch_shapes=[pltpu.VMEM((L, B, 8 * HID2), bf16)],
        compiler_params=pltpu.CompilerParams(
            dimension_semantics=("arbitrary",),
            vmem_limit_bytes=56 * 1024 * 1024),
        name="bilstm",
    )(x_tm, wih, wcat, bg, h0.astype(f32), c0.astype(f32))

    trans_pad = jnp.full((128, 128), NEG, f32).at[:NT, :NT].set(
        transitions.astype(f32))
    tcs = jnp.full((1, 128), NEG, f32).at[0, :NT].set(
        transitions[:, STOP].astype(f32))
    bout_r = jnp.zeros((1, 128), f32).at[0, :NT].set(bout.astype(f32))
    lab_prev = jnp.concatenate(
        [jnp.full((B, 1), START, tags.dtype), tags[:, :-1]], axis=1)
    tp = jnp.stack([tags, lab_prev]).astype(jnp.int8).transpose(
        0, 2, 1)[:, :, :, None]
    len_c = lengths.astype(jnp.int32)[:, None]

    parts = pl.pallas_call(
        _crf_kernel,
        grid=(1,),
        in_specs=[
            pl.BlockSpec((2, L, B, 128), lambda i: (0, 0, 0, 0)),
            pl.BlockSpec((2, L, B, 1), lambda i: (0, 0, 0, 0)),
            pl.BlockSpec((B, 1), lambda i: (0, 0)),
            pl.BlockSpec((128, 128), lambda i: (0, 0)),
            pl.BlockSpec((1, 128), lambda i: (0, 0)),
            pl.BlockSpec((1, 128), lambda i: (0, 0)),
        ],
        out_specs=pl.BlockSpec((1, 1, 128), lambda i: (0, 0, 0)),
        out_shape=jax.ShapeDtypeStruct((1, 1, 128), f32),
        scratch_shapes=[pltpu.VMEM((L, B, 128), f32)],
        compiler_params=pltpu.CompilerParams(
            dimension_semantics=("arbitrary",),
            vmem_limit_bytes=56 * 1024 * 1024),
        name="crf_nll",
    )(plog, tp, len_c, trans_pad, tcs, bout_r)

    return parts[0, 0, 0]


# speedup vs baseline: 4.2651x; 1.0692x over previous
import functools

import jax
import jax.numpy as jnp
from jax.experimental import pallas as pl
from jax.experimental.pallas import tpu as pltpu

B, L, V = 64, 256, 8000
D_EMB = 100
D_MODEL = 256
H, DK = 4, 64
HID, HID2 = 256, 128
NT = 20
START, STOP = 18, 19
NEG = -1.0e30
_DNT = (((1,), (1,)), ((), ()))

_BBA = 8
_BHALF = B // 2


def _attn_kernel(sent_ref, bush_ref, e1p_ref, e2a_ref, e2b_ref, rest_ref,
                 wq_ref, wk_ref, wv_ref, wo_ref,
                 bq_ref, bk_ref, bv_ref, bo_ref, x_ref, fin_ref):
    wq = wq_ref[...]
    wk = wk_ref[...]
    wv = wv_ref[...]
    wo = wo_ref[...]
    blk = pl.program_id(0) * (B // _BBA // 2) + pl.program_id(1)
    for ii in range(_BBA):
        item = blk * _BBA + ii
        for g in range(L // 8):
            rows0 = []
            rows1 = []
            for j in range(8):
                t = g * 8 + j
                tok1 = sent_ref[item, t]
                tok2 = bush_ref[item, t]
                rows0.append(e1p_ref[pl.ds(tok1, 1), :]
                             + e2a_ref[pl.ds(tok2, 1), :])
                rows1.append(e2b_ref[pl.ds(tok2, 1), :])
            lo = jnp.concatenate(rows0, axis=0)
            hi = (jnp.concatenate(rows1, axis=0)
                  + rest_ref[ii, g * 8:(g + 1) * 8, :])
            fin_ref[ii, g * 8:(g + 1) * 8, 0:128] = lo.astype(jnp.bfloat16)
            fin_ref[ii, g * 8:(g + 1) * 8, 128:256] = hi.astype(jnp.bfloat16)
    fall = fin_ref[...].reshape(_BBA * L, D_MODEL)
    qb = (jax.lax.dot_general(fall, wq, _DNT,
                              preferred_element_type=jnp.float32)
          + bq_ref[...]).astype(jnp.bfloat16)
    kb = (jax.lax.dot_general(fall, wk, _DNT,
                              preferred_element_type=jnp.float32)
          + bk_ref[...]).astype(jnp.bfloat16)
    vb = (jax.lax.dot_general(fall, wv, _DNT,
                              preferred_element_type=jnp.float32)
          + bv_ref[...]).astype(jnp.bfloat16)
    cats = []
    for ii in range(_BBA):
        rs = slice(ii * L, (ii + 1) * L)
        outs = []
        for h in range(H):
            sl = slice(h * DK, (h + 1) * DK)
            s = jax.lax.dot_general(
                qb[rs, sl], kb[rs, sl], _DNT,
                preferred_element_type=jnp.float32) * 0.125
            m = jnp.max(s, axis=1, keepdims=True)
            e = jnp.exp(s - m)
            l = jnp.sum(e, axis=1, keepdims=True)
            o = jnp.dot(e.astype(jnp.bfloat16), vb[rs, sl],
                        preferred_element_type=jnp.float32)
            outs.append(o / l)
        cats.append(jnp.concatenate(outs, axis=1))
    cat_all = jnp.concatenate(cats, axis=0)
    xo = jax.lax.dot_general(cat_all.astype(jnp.bfloat16), wo, _DNT,
                             preferred_element_type=jnp.float32) + bo_ref[...]
    x_ref[...] = xo.astype(jnp.bfloat16).reshape(_BBA, L, D_MODEL)


def _lstm_kernel(x_ref, wih_ref, wcat_ref, bg_ref, h0_ref, c0_ref,
                 plog_ref, xg_ref):
    for half in range(2):
        xall = x_ref[:, half * _BHALF:(half + 1) * _BHALF, :].reshape(
            L * _BHALF, D_MODEL)
        xg = jax.lax.dot_general(xall, wih_ref[...], (((1,), (1,)), ((), ())),
                                 preferred_element_type=jnp.float32)
        xg_ref[:, half * _BHALF:(half + 1) * _BHALF, :] = (
            xg.astype(jnp.bfloat16).reshape(L, _BHALF, 8 * HID2))

    wf = wcat_ref[0]
    wb = wcat_ref[1]
    G4 = 4 * HID2
    bgf = bg_ref[...][:, :G4]
    bgb = bg_ref[...][:, G4:]

    def sig(z):
        return 0.5 * jnp.tanh(0.5 * z) + 0.5

    def step(t, carry):
        hf, cf, hb, cb = carry
        tb = (L - 1) - t
        yf = jax.lax.dot_general(hf.astype(jnp.bfloat16), wf, _DNT,
                                 preferred_element_type=jnp.float32)
        yb = jax.lax.dot_general(hb.astype(jnp.bfloat16), wb, _DNT,
                                 preferred_element_type=jnp.float32)
        plog_ref[0, jnp.maximum(t - 1, 0)] = yf[:, G4:].astype(jnp.bfloat16)
        plog_ref[1, jnp.minimum(tb + 1, L - 1)] = yb[:, G4:].astype(
            jnp.bfloat16)
        gf = (xg_ref[t][:, :G4].astype(jnp.float32) + bgf) + yf[:, :G4]
        gb = (xg_ref[tb][:, G4:].astype(jnp.float32) + bgb) + yb[:, :G4]
        i_f = sig(gf[:, 0:HID2])
        f_f = sig(gf[:, HID2:2 * HID2])
        g_f = jnp.tanh(gf[:, 2 * HID2:3 * HID2])
        o_f = sig(gf[:, 3 * HID2:])
        cf = f_f * cf + i_f * g_f
        hf = o_f * jnp.tanh(cf)
        i_b = sig(gb[:, 0:HID2])
        f_b = sig(gb[:, HID2:2 * HID2])
        g_b = jnp.tanh(gb[:, 2 * HID2:3 * HID2])
        o_b = sig(gb[:, 3 * HID2:])
        cb = f_b * cb + i_b * g_b
        hb = o_b * jnp.tanh(cb)
        return hf, cf, hb, cb

    init = (h0_ref[0], c0_ref[0], h0_ref[1], c0_ref[1])
    hf, cf, hb, cb = jax.lax.fori_loop(0, L, step, init, unroll=4)
    plog_ref[0, L - 1] = jax.lax.dot_general(
        hf.astype(jnp.bfloat16), wf[G4:, :], _DNT,
        preferred_element_type=jnp.float32).astype(jnp.bfloat16)
    plog_ref[1, 0] = jax.lax.dot_general(
        hb.astype(jnp.bfloat16), wb[G4:, :], _DNT,
        preferred_element_type=jnp.float32).astype(jnp.bfloat16)


def _crf_kernel(plog_ref, tp_ref, len_ref, trans_ref, tcs_ref,
                bout_ref, out_ref, logits_ref):
    logits_ref[...] = (plog_ref[0].astype(jnp.float32)
                       + plog_ref[1].astype(jnp.float32) + bout_ref[...])

    lane3 = jax.lax.broadcasted_iota(jnp.int32, (L, B, 128), 2)
    t3 = jax.lax.broadcasted_iota(jnp.int32, (L, B, 128), 0)
    tag3 = jnp.broadcast_to(tp_ref[0].astype(jnp.int32), (L, B, 128))
    prev3 = jnp.broadcast_to(tp_ref[1].astype(jnp.int32), (L, B, 128))
    len3 = jnp.broadcast_to(len_ref[...][None, :, :], (L, B, 128))

    curoh = jnp.where(lane3 == tag3, 1.0, 0.0)
    prevoh = jnp.where(lane3 == prev3, 1.0, 0.0)
    maskf = jnp.where(t3 < len3, 1.0, 0.0)

    transb = trans_ref[...].astype(jnp.bfloat16)
    rowvals = jnp.dot(
        prevoh.astype(jnp.bfloat16).reshape(L * B, 128), transb,
        preferred_element_type=jnp.float32).reshape(L, B, 128)
    lastsel = jnp.where(t3 == (len3 - 1), 1.0, 0.0)
    acc3 = ((logits_ref[...] + rowvals) * curoh * maskf
            + lastsel * curoh * tcs_ref[...][None, :, :])
    realb = jnp.sum(jnp.sum(acc3, axis=2, keepdims=True), axis=0)

    eb = jnp.exp(trans_ref[...]).astype(jnp.bfloat16)
    lane2 = jax.lax.broadcasted_iota(jnp.int32, (B, 128), 1)
    prev0 = jnp.where(lane2 < NT, 0.0, NEG)
    lenb = len_ref[...]

    tmx = jnp.max(jnp.max(trans_ref[...], axis=1, keepdims=True),
                  axis=0, keepdims=True)
    gm = jnp.max(logits_ref[...], axis=2, keepdims=True)
    gs = jnp.maximum(gm + (tmx + 2.995732273553991)[None, :, :], 0.0)
    gprev = jnp.concatenate(
        [jnp.zeros((1, B, 1), jnp.float32), gs[:-1]], axis=0)
    logits_ref[:, :, 127:128] = gs + gprev

    def step(t, carry):
        prv, m1, m0 = carry
        lg = logits_ref[t]
        mu = m0 + lg[:, 127:128]
        p = jnp.exp(prv - mu).astype(jnp.bfloat16)
        s = jnp.dot(p, eb, preferred_element_type=jnp.float32)
        new = jnp.maximum(jnp.log(s), NEG) + mu + lg
        prvn = jnp.where(t < lenb, new, prv)
        m1n = jnp.max(prvn, axis=1, keepdims=True)
        return prvn, m1n, m1

    m_init = jnp.zeros((B, 1), jnp.float32)
    prv, _, _ = jax.lax.fori_loop(0, L, step, (prev0, m_init, m_init),
                                  unroll=4)
    v = prv + tcs_ref[...]
    m2 = jnp.max(v, axis=1, keepdims=True)
    tot = jnp.log(jnp.sum(jnp.exp(v - m2), axis=1, keepdims=True)) + m2
    part = jnp.sum(tot - realb)
    out_ref[...] = jnp.zeros((1, 1, 128), jnp.float32) + part


def kernel(sentences, bushou, pinyin, weizhi, trans_weizhi, tags, lengths,
           emb1, emb2, Wq, bq, Wk, bk, Wv, bv, Wo, bo,
           Wih_f, Whh_f, bih_f, bhh_f, Wih_b, Whh_b, bih_b, bhh_b,
           Wout, bout, transitions, h0, c0):
    f32 = jnp.float32
    bf16 = jnp.bfloat16
    e1p = jnp.pad(emb1.astype(f32), ((0, 0), (0, 128 - D_EMB)))
    e2a = jnp.pad(emb2[:, :128 - D_EMB].astype(f32), ((0, 0), (D_EMB, 0)))
    e2b = jnp.pad(emb2[:, 128 - D_EMB:].astype(f32),
                  ((0, 0), (0, 256 - 2 * D_EMB)))
    rest = jnp.pad(
        jnp.concatenate([pinyin, trans_weizhi, weizhi], axis=2).astype(f32),
        ((0, 0), (0, 0), (2 * D_EMB - 128, 0)))
    sent_i = sentences.astype(jnp.int32)
    bush_i = bushou.astype(jnp.int32)

    wq = Wq.astype(bf16)
    wk = Wk.astype(bf16)
    wv = Wv.astype(bf16)
    wo = Wo.astype(bf16)
    bq_r = bq.reshape(1, D_MODEL).astype(f32)
    bk_r = bk.reshape(1, D_MODEL).astype(f32)
    bv_r = bv.reshape(1, D_MODEL).astype(f32)
    bo_r = bo.reshape(1, D_MODEL).astype(f32)

    nblk = B // _BBA
    x_bm = pl.pallas_call(
        _attn_kernel,
        grid=(2, nblk // 2),
        in_specs=[
            pl.BlockSpec(memory_space=pltpu.SMEM),
            pl.BlockSpec(memory_space=pltpu.SMEM),
            pl.BlockSpec((V, 128), lambda c, i: (0, 0)),
            pl.BlockSpec((V, 128), lambda c, i: (0, 0)),
            pl.BlockSpec((V, 128), lambda c, i: (0, 0)),
            pl.BlockSpec((_BBA, L, 128),
                         lambda c, i: (c * (nblk // 2) + i, 0, 0)),
            pl.BlockSpec((D_MODEL, D_MODEL), lambda c, i: (0, 0)),
            pl.BlockSpec((D_MODEL, D_MODEL), lambda c, i: (0, 0)),
            pl.BlockSpec((D_MODEL, D_MODEL), lambda c, i: (0, 0)),
            pl.BlockSpec((D_MODEL, D_MODEL), lambda c, i: (0, 0)),
            pl.BlockSpec((1, D_MODEL), lambda c, i: (0, 0)),
            pl.BlockSpec((1, D_MODEL), lambda c, i: (0, 0)),
            pl.BlockSpec((1, D_MODEL), lambda c, i: (0, 0)),
            pl.BlockSpec((1, D_MODEL), lambda c, i: (0, 0)),
        ],
        out_specs=pl.BlockSpec((_BBA, L, D_MODEL),
                               lambda c, i: (c * (nblk // 2) + i, 0, 0)),
        out_shape=jax.ShapeDtypeStruct((B, L, D_MODEL), bf16),
        scratch_shapes=[pltpu.VMEM((_BBA, L, D_MODEL), bf16)],
        compiler_params=pltpu.CompilerParams(
            dimension_semantics=("parallel", "parallel"),
            vmem_limit_bytes=48 * 1024 * 1024),
        name="attn_fused",
    )(sent_i, bush_i, e1p, e2a, e2b, rest,
      wq, wk, wv, wo, bq_r, bk_r, bv_r, bo_r)

    x_tm = x_bm.transpose(1, 0, 2)

    wih = jnp.concatenate([Wih_f, Wih_b], axis=0).astype(bf16)
    bg = jnp.concatenate([(bih_f + bhh_f), (bih_b + bhh_b)]
                         ).reshape(1, 8 * HID2).astype(f32)
    woutp = jnp.zeros((2, 128, HID2), f32).at[0, :NT].set(
        Wout[:, :HID2].astype(f32)).at[1, :NT].set(Wout[:, HID2:].astype(f32))
    wcat = jnp.stack([
        jnp.concatenate([Whh_f.astype(f32), woutp[0]], axis=0),
        jnp.concatenate([Whh_b.astype(f32), woutp[1]], axis=0),
    ]).astype(bf16)

    plog = pl.pallas_call(
        _lstm_kernel,
        grid=(1,),
        in_specs=[
            pl.BlockSpec((L, B, D_MODEL), lambda i: (0, 0, 0)),
            pl.BlockSpec((8 * HID2, D_MODEL), lambda i: (0, 0)),
            pl.BlockSpec((2, 640, HID2), lambda i: (0, 0, 0)),
            pl.BlockSpec((1, 8 * HID2), lambda i: (0, 0)),
            pl.BlockSpec((2, B, HID2), lambda i: (0, 0, 0)),
            pl.BlockSpec((2, B, HID2), lambda i: (0, 0, 0)),
        ],
        out_specs=pl.BlockSpec((2, L, B, 128), lambda i: (0, 0, 0, 0)),
        out_shape=jax.ShapeDtypeStruct((2, L, B, 128), bf16),
        scratch_shapes=[pltpu.VMEM((L, B, 8 * HID2), bf16)],
        compiler_params=pltpu.CompilerParams(
            dimension_semantics=("arbitrary",),
            vmem_limit_bytes=56 * 1024 * 1024),
        name="bilstm",
    )(x_tm, wih, wcat, bg, h0.astype(f32), c0.astype(f32))

    trans_pad = jnp.full((128, 128), NEG, f32).at[:NT, :NT].set(
        transitions.astype(f32))
    tcs = jnp.full((1, 128), NEG, f32).at[0, :NT].set(
        transitions[:, STOP].astype(f32))
    bout_r = jnp.zeros((1, 128), f32).at[0, :NT].set(bout.astype(f32))
    lab_prev = jnp.concatenate(
        [jnp.full((B, 1), START, tags.dtype), tags[:, :-1]], axis=1)
    tp = jnp.stack([tags, lab_prev]).astype(jnp.int8).transpose(
        0, 2, 1)[:, :, :, None]
    len_c = lengths.astype(jnp.int32)[:, None]

    parts = pl.pallas_call(
        _crf_kernel,
        grid=(1,),
        in_specs=[
            pl.BlockSpec((2, L, B, 128), lambda i: (0, 0, 0, 0)),
            pl.BlockSpec((2, L, B, 1), lambda i: (0, 0, 0, 0)),
            pl.BlockSpec((B, 1), lambda i: (0, 0)),
            pl.BlockSpec((128, 128), lambda i: (0, 0)),
            pl.BlockSpec((1, 128), lambda i: (0, 0)),
            pl.BlockSpec((1, 128), lambda i: (0, 0)),
        ],
        out_specs=pl.BlockSpec((1, 1, 128), lambda i: (0, 0, 0)),
        out_shape=jax.ShapeDtypeStruct((1, 1, 128), f32),
        scratch_shapes=[pltpu.VMEM((L, B, 128), f32)],
        compiler_params=pltpu.CompilerParams(
            dimension_semantics=("arbitrary",),
            vmem_limit_bytes=56 * 1024 * 1024),
        name="crf_nll",
    )(plog, tp, len_c, trans_pad, tcs, bout_r)

    return parts[0, 0, 0]


# speedup vs baseline: 4.3581x; 1.0218x over previous
import functools

import jax
import jax.numpy as jnp
from jax.experimental import pallas as pl
from jax.experimental.pallas import tpu as pltpu

B, L, V = 64, 256, 8000
D_EMB = 100
D_MODEL = 256
H, DK = 4, 64
HID, HID2 = 256, 128
NT = 20
START, STOP = 18, 19
NEG = -1.0e30
_DNT = (((1,), (1,)), ((), ()))

_BBA = 8
_BHALF = B // 2


def _attn_kernel(sent_ref, bush_ref, e1p_ref, e2a_ref, e2b_ref, rest_ref,
                 wq_ref, wk_ref, wv_ref, wo_ref,
                 bq_ref, bk_ref, bv_ref, bo_ref, x_ref, fin_ref):
    wq = wq_ref[...]
    wk = wk_ref[...]
    wv = wv_ref[...]
    wo = wo_ref[...]
    blk = pl.program_id(0) * (B // _BBA // 2) + pl.program_id(1)
    for ii in range(_BBA):
        item = blk * _BBA + ii
        for g in range(L // 8):
            rows0 = []
            rows1 = []
            for j in range(8):
                t = g * 8 + j
                tok1 = sent_ref[item, t]
                tok2 = bush_ref[item, t]
                rows0.append(e1p_ref[pl.ds(tok1, 1), :]
                             + e2a_ref[pl.ds(tok2, 1), :])
                rows1.append(e2b_ref[pl.ds(tok2, 1), :])
            lo = jnp.concatenate(rows0, axis=0)
            hi = (jnp.concatenate(rows1, axis=0)
                  + rest_ref[ii, g * 8:(g + 1) * 8, :])
            fin_ref[ii, g * 8:(g + 1) * 8, 0:128] = lo.astype(jnp.bfloat16)
            fin_ref[ii, g * 8:(g + 1) * 8, 128:256] = hi.astype(jnp.bfloat16)
    fall = fin_ref[...].reshape(_BBA * L, D_MODEL)
    qb = (jax.lax.dot_general(fall, wq, _DNT,
                              preferred_element_type=jnp.float32)
          + bq_ref[...]).astype(jnp.bfloat16)
    kb = (jax.lax.dot_general(fall, wk, _DNT,
                              preferred_element_type=jnp.float32)
          + bk_ref[...]).astype(jnp.bfloat16)
    vb = (jax.lax.dot_general(fall, wv, _DNT,
                              preferred_element_type=jnp.float32)
          + bv_ref[...]).astype(jnp.bfloat16)
    cats = []
    for ii in range(_BBA):
        rs = slice(ii * L, (ii + 1) * L)
        outs = []
        for h in range(H):
            sl = slice(h * DK, (h + 1) * DK)
            s = jax.lax.dot_general(
                qb[rs, sl], kb[rs, sl], _DNT,
                preferred_element_type=jnp.float32) * 0.125
            m = jnp.max(s, axis=1, keepdims=True)
            e = jnp.exp(s - m)
            l = jnp.sum(e, axis=1, keepdims=True)
            o = jnp.dot(e.astype(jnp.bfloat16), vb[rs, sl],
                        preferred_element_type=jnp.float32)
            outs.append(o / l)
        cats.append(jnp.concatenate(outs, axis=1))
    cat_all = jnp.concatenate(cats, axis=0)
    xo = jax.lax.dot_general(cat_all.astype(jnp.bfloat16), wo, _DNT,
                             preferred_element_type=jnp.float32) + bo_ref[...]
    x_ref[...] = xo.astype(jnp.bfloat16).reshape(_BBA, L, D_MODEL)


def _lstm_kernel(x_ref, wih_ref, wcat_ref, bg_ref, h0_ref, c0_ref,
                 plog_ref, xg_ref):
    for half in range(2):
        xall = x_ref[:, half * _BHALF:(half + 1) * _BHALF, :].reshape(
            L * _BHALF, D_MODEL)
        xg = jax.lax.dot_general(xall, wih_ref[...], (((1,), (1,)), ((), ())),
                                 preferred_element_type=jnp.float32)
        xg_ref[:, half * _BHALF:(half + 1) * _BHALF, :] = (
            xg.astype(jnp.bfloat16).reshape(L, _BHALF, 8 * HID2))

    wf = wcat_ref[0]
    wb = wcat_ref[1]
    G4 = 4 * HID2
    bgf = bg_ref[...][:, :G4]
    bgb = bg_ref[...][:, G4:]

    def sig(z):
        return 0.5 * jnp.tanh(0.5 * z) + 0.5

    def step(t, carry):
        hf, cf, hb, cb = carry
        tb = (L - 1) - t
        yf = jax.lax.dot_general(hf.astype(jnp.bfloat16), wf, _DNT,
                                 preferred_element_type=jnp.float32)
        yb = jax.lax.dot_general(hb.astype(jnp.bfloat16), wb, _DNT,
                                 preferred_element_type=jnp.float32)
        plog_ref[0, jnp.maximum(t - 1, 0)] = yf[:, G4:].astype(jnp.bfloat16)
        plog_ref[1, jnp.minimum(tb + 1, L - 1)] = yb[:, G4:].astype(
            jnp.bfloat16)
        gf = (xg_ref[t][:, :G4].astype(jnp.float32) + bgf) + yf[:, :G4]
        gb = (xg_ref[tb][:, G4:].astype(jnp.float32) + bgb) + yb[:, :G4]
        i_f = sig(gf[:, 0:HID2])
        f_f = sig(gf[:, HID2:2 * HID2])
        g_f = jnp.tanh(gf[:, 2 * HID2:3 * HID2])
        o_f = sig(gf[:, 3 * HID2:])
        cf = f_f * cf + i_f * g_f
        hf = o_f * jnp.tanh(cf)
        i_b = sig(gb[:, 0:HID2])
        f_b = sig(gb[:, HID2:2 * HID2])
        g_b = jnp.tanh(gb[:, 2 * HID2:3 * HID2])
        o_b = sig(gb[:, 3 * HID2:])
        cb = f_b * cb + i_b * g_b
        hb = o_b * jnp.tanh(cb)
        return hf, cf, hb, cb

    init = (h0_ref[0], c0_ref[0], h0_ref[1], c0_ref[1])
    hf, cf, hb, cb = jax.lax.fori_loop(0, L, step, init, unroll=8)
    plog_ref[0, L - 1] = jax.lax.dot_general(
        hf.astype(jnp.bfloat16), wf[G4:, :], _DNT,
        preferred_element_type=jnp.float32).astype(jnp.bfloat16)
    plog_ref[1, 0] = jax.lax.dot_general(
        hb.astype(jnp.bfloat16), wb[G4:, :], _DNT,
        preferred_element_type=jnp.float32).astype(jnp.bfloat16)


def _crf_kernel(plog_ref, tp_ref, len_ref, trans_ref, tcs_ref,
                bout_ref, out_ref, logits_ref):
    logits_ref[...] = (plog_ref[0].astype(jnp.float32)
                       + plog_ref[1].astype(jnp.float32) + bout_ref[...])

    lane3 = jax.lax.broadcasted_iota(jnp.int32, (L, B, 128), 2)
    t3 = jax.lax.broadcasted_iota(jnp.int32, (L, B, 128), 0)
    tag3 = jnp.broadcast_to(tp_ref[0].astype(jnp.int32), (L, B, 128))
    prev3 = jnp.broadcast_to(tp_ref[1].astype(jnp.int32), (L, B, 128))
    len3 = jnp.broadcast_to(len_ref[...][None, :, :], (L, B, 128))

    curoh = jnp.where(lane3 == tag3, 1.0, 0.0)
    prevoh = jnp.where(lane3 == prev3, 1.0, 0.0)
    maskf = jnp.where(t3 < len3, 1.0, 0.0)

    transb = trans_ref[...].astype(jnp.bfloat16)
    rowvals = jnp.dot(
        prevoh.astype(jnp.bfloat16).reshape(L * B, 128), transb,
        preferred_element_type=jnp.float32).reshape(L, B, 128)
    lastsel = jnp.where(t3 == (len3 - 1), 1.0, 0.0)
    acc3 = ((logits_ref[...] + rowvals) * curoh * maskf
            + lastsel * curoh * tcs_ref[...][None, :, :])
    realb = jnp.sum(jnp.sum(acc3, axis=2, keepdims=True), axis=0)

    eb = jnp.exp(trans_ref[...]).astype(jnp.bfloat16)
    lane2 = jax.lax.broadcasted_iota(jnp.int32, (B, 128), 1)
    prev0 = jnp.where(lane2 < NT, 0.0, NEG)
    lenb = len_ref[...]

    tmx = jnp.max(jnp.max(trans_ref[...], axis=1, keepdims=True),
                  axis=0, keepdims=True)
    gm = jnp.max(logits_ref[...], axis=2, keepdims=True)
    gs = jnp.maximum(gm + (tmx + 2.995732273553991)[None, :, :], 0.0)
    gprev = jnp.concatenate(
        [jnp.zeros((1, B, 1), jnp.float32), gs[:-1]], axis=0)
    logits_ref[:, :, 127:128] = gs + gprev

    def step(t, carry):
        prv, m1, m0 = carry
        lg = logits_ref[t]
        mu = m0 + lg[:, 127:128]
        p = jnp.exp(prv - mu).astype(jnp.bfloat16)
        s = jnp.dot(p, eb, preferred_element_type=jnp.float32)
        new = jnp.maximum(jnp.log(s), NEG) + mu + lg
        prvn = jnp.where(t < lenb, new, prv)
        m1n = jnp.max(prvn, axis=1, keepdims=True)
        return prvn, m1n, m1

    m_init = jnp.zeros((B, 1), jnp.float32)
    prv, _, _ = jax.lax.fori_loop(0, L, step, (prev0, m_init, m_init),
                                  unroll=8)
    v = prv + tcs_ref[...]
    m2 = jnp.max(v, axis=1, keepdims=True)
    tot = jnp.log(jnp.sum(jnp.exp(v - m2), axis=1, keepdims=True)) + m2
    part = jnp.sum(tot - realb)
    out_ref[...] = jnp.zeros((1, 1, 128), jnp.float32) + part


def kernel(sentences, bushou, pinyin, weizhi, trans_weizhi, tags, lengths,
           emb1, emb2, Wq, bq, Wk, bk, Wv, bv, Wo, bo,
           Wih_f, Whh_f, bih_f, bhh_f, Wih_b, Whh_b, bih_b, bhh_b,
           Wout, bout, transitions, h0, c0):
    f32 = jnp.float32
    bf16 = jnp.bfloat16
    e1p = jnp.pad(emb1.astype(f32), ((0, 0), (0, 128 - D_EMB)))
    e2a = jnp.pad(emb2[:, :128 - D_EMB].astype(f32), ((0, 0), (D_EMB, 0)))
    e2b = jnp.pad(emb2[:, 128 - D_EMB:].astype(f32),
                  ((0, 0), (0, 256 - 2 * D_EMB)))
    rest = jnp.pad(
        jnp.concatenate([pinyin, trans_weizhi, weizhi], axis=2).astype(f32),
        ((0, 0), (0, 0), (2 * D_EMB - 128, 0)))
    sent_i = sentences.astype(jnp.int32)
    bush_i = bushou.astype(jnp.int32)

    wq = Wq.astype(bf16)
    wk = Wk.astype(bf16)
    wv = Wv.astype(bf16)
    wo = Wo.astype(bf16)
    bq_r = bq.reshape(1, D_MODEL).astype(f32)
    bk_r = bk.reshape(1, D_MODEL).astype(f32)
    bv_r = bv.reshape(1, D_MODEL).astype(f32)
    bo_r = bo.reshape(1, D_MODEL).astype(f32)

    nblk = B // _BBA
    x_bm = pl.pallas_call(
        _attn_kernel,
        grid=(2, nblk // 2),
        in_specs=[
            pl.BlockSpec(memory_space=pltpu.SMEM),
            pl.BlockSpec(memory_space=pltpu.SMEM),
            pl.BlockSpec((V, 128), lambda c, i: (0, 0)),
            pl.BlockSpec((V, 128), lambda c, i: (0, 0)),
            pl.BlockSpec((V, 128), lambda c, i: (0, 0)),
            pl.BlockSpec((_BBA, L, 128),
                         lambda c, i: (c * (nblk // 2) + i, 0, 0)),
            pl.BlockSpec((D_MODEL, D_MODEL), lambda c, i: (0, 0)),
            pl.BlockSpec((D_MODEL, D_MODEL), lambda c, i: (0, 0)),
            pl.BlockSpec((D_MODEL, D_MODEL), lambda c, i: (0, 0)),
            pl.BlockSpec((D_MODEL, D_MODEL), lambda c, i: (0, 0)),
            pl.BlockSpec((1, D_MODEL), lambda c, i: (0, 0)),
            pl.BlockSpec((1, D_MODEL), lambda c, i: (0, 0)),
            pl.BlockSpec((1, D_MODEL), lambda c, i: (0, 0)),
            pl.BlockSpec((1, D_MODEL), lambda c, i: (0, 0)),
        ],
        out_specs=pl.BlockSpec((_BBA, L, D_MODEL),
                               lambda c, i: (c * (nblk // 2) + i, 0, 0)),
        out_shape=jax.ShapeDtypeStruct((B, L, D_MODEL), bf16),
        scratch_shapes=[pltpu.VMEM((_BBA, L, D_MODEL), bf16)],
        compiler_params=pltpu.CompilerParams(
            dimension_semantics=("parallel", "parallel"),
            vmem_limit_bytes=48 * 1024 * 1024),
        name="attn_fused",
    )(sent_i, bush_i, e1p, e2a, e2b, rest,
      wq, wk, wv, wo, bq_r, bk_r, bv_r, bo_r)

    x_tm = x_bm.transpose(1, 0, 2)

    wih = jnp.concatenate([Wih_f, Wih_b], axis=0).astype(bf16)
    bg = jnp.concatenate([(bih_f + bhh_f), (bih_b + bhh_b)]
                         ).reshape(1, 8 * HID2).astype(f32)
    woutp = jnp.zeros((2, 128, HID2), f32).at[0, :NT].set(
        Wout[:, :HID2].astype(f32)).at[1, :NT].set(Wout[:, HID2:].astype(f32))
    wcat = jnp.stack([
        jnp.concatenate([Whh_f.astype(f32), woutp[0]], axis=0),
        jnp.concatenate([Whh_b.astype(f32), woutp[1]], axis=0),
    ]).astype(bf16)

    plog = pl.pallas_call(
        _lstm_kernel,
        grid=(1,),
        in_specs=[
            pl.BlockSpec((L, B, D_MODEL), lambda i: (0, 0, 0)),
            pl.BlockSpec((8 * HID2, D_MODEL), lambda i: (0, 0)),
            pl.BlockSpec((2, 640, HID2), lambda i: (0, 0, 0)),
            pl.BlockSpec((1, 8 * HID2), lambda i: (0, 0)),
            pl.BlockSpec((2, B, HID2), lambda i: (0, 0, 0)),
            pl.BlockSpec((2, B, HID2), lambda i: (0, 0, 0)),
        ],
        out_specs=pl.BlockSpec((2, L, B, 128), lambda i: (0, 0, 0, 0)),
        out_shape=jax.ShapeDtypeStruct((2, L, B, 128), bf16),
        scratch_shapes=[pltpu.VMEM((L, B, 8 * HID2), bf16)],
        compiler_params=pltpu.CompilerParams(
            dimension_semantics=("arbitrary",),
            vmem_limit_bytes=56 * 1024 * 1024),
        name="bilstm",
    )(x_tm, wih, wcat, bg, h0.astype(f32), c0.astype(f32))

    trans_pad = jnp.full((128, 128), NEG, f32).at[:NT, :NT].set(
        transitions.astype(f32))
    tcs = jnp.full((1, 128), NEG, f32).at[0, :NT].set(
        transitions[:, STOP].astype(f32))
    bout_r = jnp.zeros((1, 128), f32).at[0, :NT].set(bout.astype(f32))
    lab_prev = jnp.concatenate(
        [jnp.full((B, 1), START, tags.dtype), tags[:, :-1]], axis=1)
    tp = jnp.stack([tags, lab_prev]).astype(jnp.int8).transpose(
        0, 2, 1)[:, :, :, None]
    len_c = lengths.astype(jnp.int32)[:, None]

    parts = pl.pallas_call(
        _crf_kernel,
        grid=(1,),
        in_specs=[
            pl.BlockSpec((2, L, B, 128), lambda i: (0, 0, 0, 0)),
            pl.BlockSpec((2, L, B, 1), lambda i: (0, 0, 0, 0)),
            pl.BlockSpec((B, 1), lambda i: (0, 0)),
            pl.BlockSpec((128, 128), lambda i: (0, 0)),
            pl.BlockSpec((1, 128), lambda i: (0, 0)),
            pl.BlockSpec((1, 128), lambda i: (0, 0)),
        ],
        out_specs=pl.BlockSpec((1, 1, 128), lambda i: (0, 0, 0)),
        out_shape=jax.ShapeDtypeStruct((1, 1, 128), f32),
        scratch_shapes=[pltpu.VMEM((L, B, 128), f32)],
        compiler_params=pltpu.CompilerParams(
            dimension_semantics=("arbitrary",),
            vmem_limit_bytes=56 * 1024 * 1024),
        name="crf_nll",
    )(plog, tp, len_c, trans_pad, tcs, bout_r)

    return parts[0, 0, 0]
